# Initial kernel scaffold; baseline (speedup 1.0000x reference)
import functools

import jax
import jax.numpy as jnp
from jax import lax
from jax.experimental import pallas as pl
from jax.experimental.pallas import tpu as pltpu
from jax.experimental.pallas import tpu_sc as plsc

N = 50000
E = 800000
F = 96

NCORE = 2
NSUB = 16
NW = NCORE * NSUB
EPW = E // NW
EB = 125
NBLK = EPW // EB
NBUF = 4
ZBLK = 625
NZB = N // ZBLK
ZPT = NZB // NSUB

FC = 32
NCHUNK = F // FC

RB = 1000
NRB = N // RB


def _sc_mesh():
  return plsc.VectorSubcoreMesh(core_axis_name="c", subcore_axis_name="s")


def _edge_pass(table_h, src_v, dst_v, rows, acc, gsem, ssem, gather):
  def group(g, carry):
    gdescs = []
    if gather:
      for b in range(NBUF):
        j = g * NBUF + b
        gdescs.append(
            pltpu.async_copy(table_h.at[src_v.at[j]], rows.at[b], gsem.at[b]))
      for b in range(NBUF):
        gdescs[b].wait()
    sdescs = []
    for b in range(NBUF):
      j = g * NBUF + b
      sdescs.append(
          pltpu.async_copy(rows.at[b], acc.at[dst_v.at[j]], ssem.at[b],
                           add=True))
    for b in range(NBUF):
      sdescs[b].wait()
    return carry

  lax.fori_loop(0, NBLK // NBUF, group, 0)


def _zero_acc(z_h, acc, sid):
  for t in range(ZPT):
    blk = sid + NSUB * t
    pltpu.sync_copy(z_h, acc.at[pl.ds(blk * ZBLK, ZBLK)])


def _writeout(acc, out_h, cid, sid):
  for t in range(ZPT):
    blk = sid + NSUB * t
    pltpu.sync_copy(acc.at[pl.ds(blk * ZBLK, ZBLK)],
                    out_h.at[cid, pl.ds(blk * ZBLK, ZBLK)])


def _sc_segsum_wide(y0, y1, y2, src, dst, zblk):
  out = jax.ShapeDtypeStruct((NCORE, N, FC), jnp.float32)

  @functools.partial(
      pl.kernel,
      out_type=(out, out, out),
      mesh=_sc_mesh(),
      scratch_types=[
          pltpu.VMEM((NBLK, EB), jnp.int32),
          pltpu.VMEM((NBLK, EB), jnp.int32),
          pltpu.VMEM((NBUF, EB, FC), jnp.float32),
          pltpu.VMEM_SHARED((N, FC), jnp.float32),
          pltpu.SemaphoreType.DMA((NBUF,)),
          pltpu.SemaphoreType.DMA((NBUF,)),
      ],
  )
  def k(y0_h, y1_h, y2_h, src_h, dst_h, z_h, o0_h, o1_h, o2_h,
        src_v, dst_v, rows, acc, gsem, ssem):
    cid = lax.axis_index("c")
    sid = lax.axis_index("s")
    wid = cid * NSUB + sid
    pltpu.sync_copy(src_h.at[wid], src_v)
    pltpu.sync_copy(dst_h.at[wid], dst_v)
    for y_h, o_h in ((y0_h, o0_h), (y1_h, o1_h), (y2_h, o2_h)):
      _zero_acc(z_h, acc, sid)
      plsc.subcore_barrier()
      _edge_pass(y_h, src_v, dst_v, rows, acc, gsem, ssem, gather=True)
      plsc.subcore_barrier()
      _writeout(acc, o_h, cid, sid)
      plsc.subcore_barrier()

  return k(y0, y1, y2, src, dst, zblk)


def _sc_segsum_narrow(y, src, dst, zblk):
  @functools.partial(
      pl.kernel,
      out_type=jax.ShapeDtypeStruct((NCORE, N, 8), jnp.float32),
      mesh=_sc_mesh(),
      scratch_types=[
          pltpu.VMEM((NBLK, EB), jnp.int32),
          pltpu.VMEM((NBLK, EB), jnp.int32),
          pltpu.VMEM((NBUF, EB, 8), jnp.float32),
          pltpu.VMEM_SHARED((N, 8), jnp.float32),
          pltpu.SemaphoreType.DMA((NBUF,)),
          pltpu.SemaphoreType.DMA((NBUF,)),
      ],
  )
  def k(y_h, src_h, dst_h, z_h, o_h, src_v, dst_v, rows, acc, gsem, ssem):
    cid = lax.axis_index("c")
    sid = lax.axis_index("s")
    wid = cid * NSUB + sid
    pltpu.sync_copy(src_h.at[wid], src_v)
    pltpu.sync_copy(dst_h.at[wid], dst_v)
    _zero_acc(z_h, acc, sid)
    plsc.subcore_barrier()
    _edge_pass(y_h, src_v, dst_v, rows, acc, gsem, ssem, gather=True)
    plsc.subcore_barrier()
    _writeout(acc, o_h, cid, sid)

  return k(y, src, dst, zblk)


def _sc_count(ones_rows, dst, zblk):
  @functools.partial(
      pl.kernel,
      out_type=jax.ShapeDtypeStruct((NCORE, N, 8), jnp.float32),
      mesh=_sc_mesh(),
      scratch_types=[
          pltpu.VMEM((NBLK, EB), jnp.int32),
          pltpu.VMEM((NBUF, EB, 8), jnp.float32),
          pltpu.VMEM_SHARED((N, 8), jnp.float32),
          pltpu.SemaphoreType.DMA((NBUF,)),
          pltpu.SemaphoreType.DMA((NBUF,)),
      ],
  )
  def k(ones_h, dst_h, z_h, o_h, dst_v, rows, acc, gsem, ssem):
    cid = lax.axis_index("c")
    sid = lax.axis_index("s")
    wid = cid * NSUB + sid
    pltpu.sync_copy(dst_h.at[wid], dst_v)
    for b in range(NBUF):
      pltpu.sync_copy(ones_h, rows.at[b])
    _zero_acc(z_h, acc, sid)
    plsc.subcore_barrier()
    _edge_pass(None, None, dst_v, rows, acc, gsem, ssem, gather=False)
    plsc.subcore_barrier()
    _writeout(acc, o_h, cid, sid)

  return k(ones_rows, dst, zblk)


_DOT = functools.partial(
    lax.dot_general,
    dimension_numbers=(((1,), (1,)), ((), ())),
    preferred_element_type=jnp.float32,
)

_W_SPEC = pl.BlockSpec((F, F), lambda i: (0, 0))
_B_SPEC = pl.BlockSpec((8, F), lambda i: (0, 0))
_H_SPEC = pl.BlockSpec((RB, F), lambda i: (i, 0))
_C_SPEC = pl.BlockSpec((RB, FC), lambda i: (i, 0))
_P_SPEC = pl.BlockSpec((NCORE, RB, FC), lambda i: (0, i, 0))
_P8_SPEC = pl.BlockSpec((NCORE, RB, 8), lambda i: (0, i, 0))


def _emit_layer(h, wl_ref, wr_ref, b_ref, c0_ref, c1_ref, c2_ref, yr_ref):
  yl = _DOT(h, wl_ref[...])
  c0_ref[...] = yl[:, 0:FC]
  c1_ref[...] = yl[:, FC:2 * FC]
  c2_ref[...] = yl[:, 2 * FC:3 * FC]
  yr_ref[...] = _DOT(h, wr_ref[...]) + b_ref[0:1, :]


def _tc_first(h0, wl, wr, b):
  def body(h_ref, wl_ref, wr_ref, b_ref, c0_ref, c1_ref, c2_ref, yr_ref):
    _emit_layer(h_ref[...], wl_ref, wr_ref, b_ref, c0_ref, c1_ref, c2_ref,
                yr_ref)

  cs = jax.ShapeDtypeStruct((N, FC), jnp.float32)
  return pl.pallas_call(
      body,
      grid=(NRB,),
      in_specs=[_H_SPEC, _W_SPEC, _W_SPEC, _B_SPEC],
      out_specs=[_C_SPEC, _C_SPEC, _C_SPEC, _H_SPEC],
      out_shape=[cs, cs, cs, jax.ShapeDtypeStruct((N, F), jnp.float32)],
  )(h0, wl, wr, b)


def _combine(p0_ref, p1_ref, p2_ref, cnt_ref, yr_ref):
  seg = jnp.concatenate(
      [p0_ref[0] + p0_ref[1], p1_ref[0] + p1_ref[1], p2_ref[0] + p2_ref[1]],
      axis=1)
  cnt = cnt_ref[0, :, 0:1] + cnt_ref[1, :, 0:1]
  inv = 1.0 / jnp.maximum(cnt, 1.0)
  return jax.nn.relu(seg * inv + yr_ref[...])


def _tc_mid(p0, p1, p2, cntp, yr_prev, wl, wr, b):
  def body(p0_ref, p1_ref, p2_ref, cnt_ref, yrp_ref, wl_ref, wr_ref, b_ref,
           c0_ref, c1_ref, c2_ref, yr_ref):
    h = _combine(p0_ref, p1_ref, p2_ref, cnt_ref, yrp_ref)
    _emit_layer(h, wl_ref, wr_ref, b_ref, c0_ref, c1_ref, c2_ref, yr_ref)

  cs = jax.ShapeDtypeStruct((N, FC), jnp.float32)
  return pl.pallas_call(
      body,
      grid=(NRB,),
      in_specs=[_P_SPEC, _P_SPEC, _P_SPEC, _P8_SPEC, _H_SPEC, _W_SPEC,
                _W_SPEC, _B_SPEC],
      out_specs=[_C_SPEC, _C_SPEC, _C_SPEC, _H_SPEC],
      out_shape=[cs, cs, cs, jax.ShapeDtypeStruct((N, F), jnp.float32)],
  )(p0, p1, p2, cntp, yr_prev, wl, wr, b)


def _tc_last_pre(p0, p1, p2, cntp, yr_prev, wl3p, wr3, b3):
  def body(p0_ref, p1_ref, p2_ref, cnt_ref, yrp_ref, wl_ref, wr_ref, b_ref,
           y3_ref, yr3_ref):
    h = _combine(p0_ref, p1_ref, p2_ref, cnt_ref, yrp_ref)
    y3_ref[...] = _DOT(h, wl_ref[...])
    yr3_ref[...] = _DOT(h, wr_ref[...]) + b_ref[0, 0]

  return pl.pallas_call(
      body,
      grid=(NRB,),
      in_specs=[_P_SPEC, _P_SPEC, _P_SPEC, _P8_SPEC, _H_SPEC,
                pl.BlockSpec((8, F), lambda i: (0, 0)),
                pl.BlockSpec((1, F), lambda i: (0, 0)),
                pl.BlockSpec(memory_space=pltpu.SMEM)],
      out_specs=[pl.BlockSpec((RB, 8), lambda i: (i, 0)),
                 pl.BlockSpec((RB, 1), lambda i: (i, 0))],
      out_shape=[jax.ShapeDtypeStruct((N, 8), jnp.float32),
                 jax.ShapeDtypeStruct((N, 1), jnp.float32)],
  )(p0, p1, p2, cntp, yr_prev, wl3p, wr3, b3)


def _tc_final(p3, cntp, yr3):
  def body(p3_ref, cnt_ref, yr3_ref, out_ref, util_ref, acc_ref):
    i = pl.program_id(0)

    @pl.when(i == 0)
    def _():
      acc_ref[0] = 0.0

    seg = p3_ref[0, :, 0:1] + p3_ref[1, :, 0:1]
    cnt = cnt_ref[0, :, 0:1] + cnt_ref[1, :, 0:1]
    inv = 1.0 / jnp.maximum(cnt, 1.0)
    o = jax.nn.sigmoid(seg * inv + yr3_ref[...])
    out_ref[...] = o
    acc_ref[0] = acc_ref[0] + jnp.sum(o)
    util_ref[0] = acc_ref[0] * (1.0 / N)

  return pl.pallas_call(
      body,
      grid=(NRB,),
      in_specs=[_P8_SPEC, _P8_SPEC, pl.BlockSpec((RB, 1), lambda i: (i, 0))],
      out_specs=[pl.BlockSpec((RB, 1), lambda i: (i, 0)),
                 pl.BlockSpec(memory_space=pltpu.SMEM)],
      out_shape=[jax.ShapeDtypeStruct((N, 1), jnp.float32),
                 jax.ShapeDtypeStruct((1,), jnp.float32)],
      scratch_shapes=[pltpu.SMEM((1,), jnp.float32)],
  )(p3, cntp, yr3)


def kernel(x, diff, rec_prev, hidden_prev, edge_index,
           Wl0, Wr0, b0, Wl1, Wr1, b1, Wl2, Wr2, b2, Wl3, Wr3, b3):
  h0 = jnp.concatenate([x, diff, rec_prev, hidden_prev], axis=1)
  src = edge_index[0].reshape(NW, NBLK, EB)
  dst = edge_index[1].reshape(NW, NBLK, EB)

  z32 = jnp.zeros((ZBLK, FC), jnp.float32)
  z8 = jnp.zeros((ZBLK, 8), jnp.float32)
  ones8 = jnp.ones((EB, 8), jnp.float32)

  def pad_b(b):
    return jnp.broadcast_to(b[None, :], (8, F))

  cntp = _sc_count(ones8, dst, z8)

  c0, c1, c2, yr = _tc_first(h0, Wl0, Wr0, pad_b(b0))
  p0, p1, p2 = _sc_segsum_wide(c0, c1, c2, src, dst, z32)

  c0, c1, c2, yr = _tc_mid(p0, p1, p2, cntp, yr, Wl1, Wr1, pad_b(b1))
  p0, p1, p2 = _sc_segsum_wide(c0, c1, c2, src, dst, z32)

  c0, c1, c2, yr = _tc_mid(p0, p1, p2, cntp, yr, Wl2, Wr2, pad_b(b2))
  p0, p1, p2 = _sc_segsum_wide(c0, c1, c2, src, dst, z32)

  wl3p = jnp.concatenate([Wl3, jnp.zeros((7, F), jnp.float32)], axis=0)
  y3p, yr3 = _tc_last_pre(p0, p1, p2, cntp, yr, wl3p, Wr3,
                          b3.reshape(1, 1))
  p3 = _sc_segsum_narrow(y3p, src, dst, z8)

  out, util = _tc_final(p3, cntp, yr3)
  return out, util[0]


# speedup vs baseline: 5.2695x; 5.2695x over previous
import functools

import jax
import jax.numpy as jnp
from jax import lax
from jax.experimental import pallas as pl
from jax.experimental.pallas import tpu as pltpu
from jax.experimental.pallas import tpu_sc as plsc

N = 50000
E = 800000
F = 96

NCORE = 2
NSUB = 16
NW = NCORE * NSUB
EPW = E // NW
EB = 125
NBLK = EPW // EB
NBUF = 4
ZBLK = 1000
NZB = N // ZBLK
ZPT = -(-NZB // NSUB)

FC = 16
NCHUNK = F // FC

RB = 1000
NRB = N // RB


def _sc_mesh():
  return plsc.VectorSubcoreMesh(core_axis_name="c", subcore_axis_name="s")


_SC_PARAMS = pltpu.CompilerParams(use_tc_tiling_on_sc=False)


def _edge_pass(table_h, src_v, dst_v, rows, acc, gsem, ssem, gather):
  def group(g, carry):
    gdescs = []
    if gather:
      for b in range(NBUF):
        j = g * NBUF + b
        gdescs.append(
            pltpu.async_copy(table_h.at[src_v.at[j]], rows.at[b], gsem.at[b]))
      for b in range(NBUF):
        gdescs[b].wait()
    sdescs = []
    for b in range(NBUF):
      j = g * NBUF + b
      sdescs.append(
          pltpu.async_copy(rows.at[b], acc.at[dst_v.at[j]], ssem.at[b],
                           add=True))
    for b in range(NBUF):
      sdescs[b].wait()
    return carry

  lax.fori_loop(0, NBLK // NBUF, group, 0)


def _zero_acc(z_h, acc, sid):
  for t in range(ZPT):
    blk = sid + NSUB * t

    @pl.when(blk < NZB)
    def _():
      pltpu.sync_copy(z_h, acc.at[pl.ds(blk * ZBLK, ZBLK)])


def _writeout(acc, out_h, cid, sid):
  for t in range(ZPT):
    blk = sid + NSUB * t

    @pl.when(blk < NZB)
    def _():
      pltpu.sync_copy(acc.at[pl.ds(blk * ZBLK, ZBLK)],
                      out_h.at[cid, pl.ds(blk * ZBLK, ZBLK)])


def _sc_segsum_wide(ys, src, dst, zblk):
  out = jax.ShapeDtypeStruct((NCORE, N, FC), jnp.float32)

  @functools.partial(
      pl.kernel,
      out_type=(out,) * NCHUNK,
      mesh=_sc_mesh(),
      compiler_params=_SC_PARAMS,
      scratch_types=[
          pltpu.VMEM((NBLK, EB), jnp.int32),
          pltpu.VMEM((NBLK, EB), jnp.int32),
          pltpu.VMEM((NBUF, EB, FC), jnp.float32),
          pltpu.VMEM_SHARED((N, FC), jnp.float32),
          pltpu.SemaphoreType.DMA((NBUF,)),
          pltpu.SemaphoreType.DMA((NBUF,)),
      ],
  )
  def k(*refs):
    y_hs = refs[:NCHUNK]
    src_h, dst_h, z_h = refs[NCHUNK:NCHUNK + 3]
    o_hs = refs[NCHUNK + 3:2 * NCHUNK + 3]
    src_v, dst_v, rows, acc, gsem, ssem = refs[2 * NCHUNK + 3:]
    cid = lax.axis_index("c")
    sid = lax.axis_index("s")
    wid = cid * NSUB + sid
    pltpu.sync_copy(src_h.at[wid], src_v)
    pltpu.sync_copy(dst_h.at[wid], dst_v)
    for y_h, o_h in zip(y_hs, o_hs):
      _zero_acc(z_h, acc, sid)
      plsc.subcore_barrier()
      _edge_pass(y_h, src_v, dst_v, rows, acc, gsem, ssem, gather=True)
      plsc.subcore_barrier()
      _writeout(acc, o_h, cid, sid)
      plsc.subcore_barrier()

  return k(*ys, src, dst, zblk)


def _sc_segsum_narrow(y, src, dst, zblk):
  @functools.partial(
      pl.kernel,
      out_type=jax.ShapeDtypeStruct((NCORE, N, 8), jnp.float32),
      mesh=_sc_mesh(),
      compiler_params=_SC_PARAMS,
      scratch_types=[
          pltpu.VMEM((NBLK, EB), jnp.int32),
          pltpu.VMEM((NBLK, EB), jnp.int32),
          pltpu.VMEM((NBUF, EB, 8), jnp.float32),
          pltpu.VMEM_SHARED((N, 8), jnp.float32),
          pltpu.SemaphoreType.DMA((NBUF,)),
          pltpu.SemaphoreType.DMA((NBUF,)),
      ],
  )
  def k(y_h, src_h, dst_h, z_h, o_h, src_v, dst_v, rows, acc, gsem, ssem):
    cid = lax.axis_index("c")
    sid = lax.axis_index("s")
    wid = cid * NSUB + sid
    pltpu.sync_copy(src_h.at[wid], src_v)
    pltpu.sync_copy(dst_h.at[wid], dst_v)
    _zero_acc(z_h, acc, sid)
    plsc.subcore_barrier()
    _edge_pass(y_h, src_v, dst_v, rows, acc, gsem, ssem, gather=True)
    plsc.subcore_barrier()
    _writeout(acc, o_h, cid, sid)

  return k(y, src, dst, zblk)


def _sc_count(ones_rows, dst, zblk):
  @functools.partial(
      pl.kernel,
      out_type=jax.ShapeDtypeStruct((NCORE, N, 8), jnp.float32),
      mesh=_sc_mesh(),
      compiler_params=_SC_PARAMS,
      scratch_types=[
          pltpu.VMEM((NBLK, EB), jnp.int32),
          pltpu.VMEM((NBUF, EB, 8), jnp.float32),
          pltpu.VMEM_SHARED((N, 8), jnp.float32),
          pltpu.SemaphoreType.DMA((NBUF,)),
          pltpu.SemaphoreType.DMA((NBUF,)),
      ],
  )
  def k(ones_h, dst_h, z_h, o_h, dst_v, rows, acc, gsem, ssem):
    cid = lax.axis_index("c")
    sid = lax.axis_index("s")
    wid = cid * NSUB + sid
    pltpu.sync_copy(dst_h.at[wid], dst_v)
    for b in range(NBUF):
      pltpu.sync_copy(ones_h, rows.at[b])
    _zero_acc(z_h, acc, sid)
    plsc.subcore_barrier()
    _edge_pass(None, None, dst_v, rows, acc, gsem, ssem, gather=False)
    plsc.subcore_barrier()
    _writeout(acc, o_h, cid, sid)

  return k(ones_rows, dst, zblk)


_DOT = functools.partial(
    lax.dot_general,
    dimension_numbers=(((1,), (1,)), ((), ())),
    preferred_element_type=jnp.float32,
)

_W_SPEC = pl.BlockSpec((F, F), lambda i: (0, 0))
_B_SPEC = pl.BlockSpec((8, F), lambda i: (0, 0))
_H_SPEC = pl.BlockSpec((RB, F), lambda i: (i, 0))
_C_SPEC = pl.BlockSpec((RB, FC), lambda i: (i, 0))
_P_SPEC = pl.BlockSpec((NCORE, RB, FC), lambda i: (0, i, 0))
_P8_SPEC = pl.BlockSpec((NCORE, RB, 8), lambda i: (0, i, 0))


def _emit_layer(h, wl_ref, wr_ref, b_ref, c_refs, yr_ref):
  yl = _DOT(h, wl_ref[...])
  for c, c_ref in enumerate(c_refs):
    c_ref[...] = yl[:, c * FC:(c + 1) * FC]
  yr_ref[...] = _DOT(h, wr_ref[...]) + b_ref[0:1, :]


def _tc_first(h0, wl, wr, b):
  def body(h_ref, wl_ref, wr_ref, b_ref, *outs):
    _emit_layer(h_ref[...], wl_ref, wr_ref, b_ref, outs[:NCHUNK], outs[NCHUNK])

  cs = jax.ShapeDtypeStruct((N, FC), jnp.float32)
  return pl.pallas_call(
      body,
      grid=(NRB,),
      in_specs=[_H_SPEC, _W_SPEC, _W_SPEC, _B_SPEC],
      out_specs=[_C_SPEC] * NCHUNK + [_H_SPEC],
      out_shape=[cs] * NCHUNK + [jax.ShapeDtypeStruct((N, F), jnp.float32)],
  )(h0, wl, wr, b)


def _combine(p_refs, cnt_ref, yr_ref):
  seg = jnp.concatenate([p[0] + p[1] for p in p_refs], axis=1)
  cnt = cnt_ref[0, :, 0:1] + cnt_ref[1, :, 0:1]
  inv = 1.0 / jnp.maximum(cnt, 1.0)
  return jax.nn.relu(seg * inv + yr_ref[...])


def _tc_mid(ps, cntp, yr_prev, wl, wr, b):
  def body(*refs):
    p_refs = refs[:NCHUNK]
    cnt_ref, yrp_ref, wl_ref, wr_ref, b_ref = refs[NCHUNK:NCHUNK + 5]
    outs = refs[NCHUNK + 5:]
    h = _combine(p_refs, cnt_ref, yrp_ref)
    _emit_layer(h, wl_ref, wr_ref, b_ref, outs[:NCHUNK], outs[NCHUNK])

  cs = jax.ShapeDtypeStruct((N, FC), jnp.float32)
  return pl.pallas_call(
      body,
      grid=(NRB,),
      in_specs=[_P_SPEC] * NCHUNK + [_P8_SPEC, _H_SPEC, _W_SPEC, _W_SPEC,
                                     _B_SPEC],
      out_specs=[_C_SPEC] * NCHUNK + [_H_SPEC],
      out_shape=[cs] * NCHUNK + [jax.ShapeDtypeStruct((N, F), jnp.float32)],
  )(*ps, cntp, yr_prev, wl, wr, b)


def _tc_last_pre(ps, cntp, yr_prev, w3p, b3):
  def body(*refs):
    p_refs = refs[:NCHUNK]
    cnt_ref, yrp_ref, w_ref, b_ref = refs[NCHUNK:NCHUNK + 4]
    y3_ref, yr3_ref = refs[NCHUNK + 4:]
    h = _combine(p_refs, cnt_ref, yrp_ref)
    y3 = _DOT(h, w_ref[...])
    y3_ref[...] = y3
    yr3_ref[...] = y3[:, 1:2] + b_ref[0, 0]

  return pl.pallas_call(
      body,
      grid=(NRB,),
      in_specs=[_P_SPEC] * NCHUNK + [
          _P8_SPEC, _H_SPEC,
          pl.BlockSpec((8, F), lambda i: (0, 0)),
          pl.BlockSpec(memory_space=pltpu.SMEM)],
      out_specs=[pl.BlockSpec((RB, 8), lambda i: (i, 0)),
                 pl.BlockSpec((RB, 1), lambda i: (i, 0))],
      out_shape=[jax.ShapeDtypeStruct((N, 8), jnp.float32),
                 jax.ShapeDtypeStruct((N, 1), jnp.float32)],
  )(*ps, cntp, yr_prev, w3p, b3)


def _tc_final(p3, cntp, yr3):
  def body(p3_ref, cnt_ref, yr3_ref, out_ref, util_ref, acc_ref):
    i = pl.program_id(0)

    @pl.when(i == 0)
    def _():
      acc_ref[0] = 0.0

    seg = p3_ref[0, :, 0:1] + p3_ref[1, :, 0:1]
    cnt = cnt_ref[0, :, 0:1] + cnt_ref[1, :, 0:1]
    inv = 1.0 / jnp.maximum(cnt, 1.0)
    o = jax.nn.sigmoid(seg * inv + yr3_ref[...])
    out_ref[...] = o
    acc_ref[0] = acc_ref[0] + jnp.sum(o)
    util_ref[0] = acc_ref[0] * (1.0 / N)

  return pl.pallas_call(
      body,
      grid=(NRB,),
      in_specs=[_P8_SPEC, _P8_SPEC, pl.BlockSpec((RB, 1), lambda i: (i, 0))],
      out_specs=[pl.BlockSpec((RB, 1), lambda i: (i, 0)),
                 pl.BlockSpec(memory_space=pltpu.SMEM)],
      out_shape=[jax.ShapeDtypeStruct((N, 1), jnp.float32),
                 jax.ShapeDtypeStruct((1,), jnp.float32)],
      scratch_shapes=[pltpu.SMEM((1,), jnp.float32)],
  )(p3, cntp, yr3)


def kernel(x, diff, rec_prev, hidden_prev, edge_index,
           Wl0, Wr0, b0, Wl1, Wr1, b1, Wl2, Wr2, b2, Wl3, Wr3, b3):
  h0 = jnp.concatenate([x, diff, rec_prev, hidden_prev], axis=1)
  src = edge_index[0].reshape(NW, NBLK, EB)
  dst = edge_index[1].reshape(NW, NBLK, EB)

  zc = jnp.zeros((ZBLK, FC), jnp.float32)
  z8 = jnp.zeros((ZBLK, 8), jnp.float32)
  ones8 = jnp.ones((EB, 8), jnp.float32)

  def pad_b(b):
    return jnp.broadcast_to(b[None, :], (8, F))

  cntp = _sc_count(ones8, dst, z8)

  *cs, yr = _tc_first(h0, Wl0, Wr0, pad_b(b0))
  ps = _sc_segsum_wide(cs, src, dst, zc)

  *cs, yr = _tc_mid(ps, cntp, yr, Wl1, Wr1, pad_b(b1))
  ps = _sc_segsum_wide(cs, src, dst, zc)

  *cs, yr = _tc_mid(ps, cntp, yr, Wl2, Wr2, pad_b(b2))
  ps = _sc_segsum_wide(cs, src, dst, zc)

  w3p = jnp.concatenate([Wl3, Wr3, jnp.zeros((6, F), jnp.float32)], axis=0)
  y3p, yr3 = _tc_last_pre(ps, cntp, yr, w3p, b3.reshape(1, 1))
  p3 = _sc_segsum_narrow(y3p, src, dst, z8)

  out, util = _tc_final(p3, cntp, yr3)
  return out, util[0]


# speedup vs baseline: 5.6922x; 1.0802x over previous
import functools

import jax
import jax.numpy as jnp
from jax import lax
from jax.experimental import pallas as pl
from jax.experimental.pallas import tpu as pltpu
from jax.experimental.pallas import tpu_sc as plsc

N = 50000
E = 800000
F = 96

NCORE = 2
NSUB = 16
NW = NCORE * NSUB
EPW = E // NW
EB = 125
NBLK = EPW // EB
NBUF = 4
ZBLK = 1000
NZB = N // ZBLK
ZPT = -(-NZB // NSUB)

FC = 16
NCHUNK = F // FC

RB = 1000
NRB = N // RB


def _sc_mesh():
  return plsc.VectorSubcoreMesh(core_axis_name="c", subcore_axis_name="s")


_SC_PARAMS = pltpu.CompilerParams(use_tc_tiling_on_sc=False)


def _edge_pass(table_h, src_v, dst_v, rows, acc, gsem, ssem, gather):
  def group(g, carry):
    if gather:
      for b in range(NBUF):
        j = g * NBUF + b

        @pl.when(g > 0)
        def _():
          pltpu.make_async_copy(rows.at[b], acc.at[dst_v.at[j]],
                                ssem.at[b]).wait()

        pltpu.async_copy(table_h.at[src_v.at[j]], rows.at[b], gsem.at[b])
    gdescs = []
    sdescs = []
    for b in range(NBUF):
      j = g * NBUF + b
      if gather:
        pltpu.make_async_copy(table_h.at[src_v.at[j]], rows.at[b],
                              gsem.at[b]).wait()
      else:
        @pl.when(g > 0)
        def _():
          pltpu.make_async_copy(rows.at[b], acc.at[dst_v.at[j]],
                                ssem.at[b]).wait()
      pltpu.async_copy(rows.at[b], acc.at[dst_v.at[j]], ssem.at[b], add=True)
    return carry

  lax.fori_loop(0, NBLK // NBUF, group, 0)
  for b in range(NBUF):
    j = NBLK - NBUF + b
    pltpu.make_async_copy(rows.at[b], acc.at[dst_v.at[j]], ssem.at[b]).wait()


def _zero_acc(z_h, acc, sid):
  for t in range(ZPT):
    blk = sid + NSUB * t

    @pl.when(blk < NZB)
    def _():
      pltpu.sync_copy(z_h, acc.at[pl.ds(blk * ZBLK, ZBLK)])


def _writeout(acc, out_h, cid, sid):
  for t in range(ZPT):
    blk = sid + NSUB * t

    @pl.when(blk < NZB)
    def _():
      pltpu.sync_copy(acc.at[pl.ds(blk * ZBLK, ZBLK)],
                      out_h.at[cid, pl.ds(blk * ZBLK, ZBLK)])


def _sc_segsum_wide(ys, src, dst, zblk):
  out = jax.ShapeDtypeStruct((NCORE, N, FC), jnp.float32)

  @functools.partial(
      pl.kernel,
      out_type=(out,) * NCHUNK,
      mesh=_sc_mesh(),
      compiler_params=_SC_PARAMS,
      scratch_types=[
          pltpu.VMEM((NBLK, EB), jnp.int32),
          pltpu.VMEM((NBLK, EB), jnp.int32),
          pltpu.VMEM((NBUF, EB, FC), jnp.float32),
          pltpu.VMEM_SHARED((N, FC), jnp.float32),
          pltpu.SemaphoreType.DMA((NBUF,)),
          pltpu.SemaphoreType.DMA((NBUF,)),
      ],
  )
  def k(*refs):
    y_hs = refs[:NCHUNK]
    src_h, dst_h, z_h = refs[NCHUNK:NCHUNK + 3]
    o_hs = refs[NCHUNK + 3:2 * NCHUNK + 3]
    src_v, dst_v, rows, acc, gsem, ssem = refs[2 * NCHUNK + 3:]
    cid = lax.axis_index("c")
    sid = lax.axis_index("s")
    wid = cid * NSUB + sid
    pltpu.sync_copy(src_h.at[wid], src_v)
    pltpu.sync_copy(dst_h.at[wid], dst_v)
    for y_h, o_h in zip(y_hs, o_hs):
      _zero_acc(z_h, acc, sid)
      plsc.subcore_barrier()
      _edge_pass(y_h, src_v, dst_v, rows, acc, gsem, ssem, gather=True)
      plsc.subcore_barrier()
      _writeout(acc, o_h, cid, sid)
      plsc.subcore_barrier()

  return k(*ys, src, dst, zblk)


def _sc_segsum_narrow(y, src, dst, zblk):
  @functools.partial(
      pl.kernel,
      out_type=jax.ShapeDtypeStruct((NCORE, N, 8), jnp.float32),
      mesh=_sc_mesh(),
      compiler_params=_SC_PARAMS,
      scratch_types=[
          pltpu.VMEM((NBLK, EB), jnp.int32),
          pltpu.VMEM((NBLK, EB), jnp.int32),
          pltpu.VMEM((NBUF, EB, 8), jnp.float32),
          pltpu.VMEM_SHARED((N, 8), jnp.float32),
          pltpu.SemaphoreType.DMA((NBUF,)),
          pltpu.SemaphoreType.DMA((NBUF,)),
      ],
  )
  def k(y_h, src_h, dst_h, z_h, o_h, src_v, dst_v, rows, acc, gsem, ssem):
    cid = lax.axis_index("c")
    sid = lax.axis_index("s")
    wid = cid * NSUB + sid
    pltpu.sync_copy(src_h.at[wid], src_v)
    pltpu.sync_copy(dst_h.at[wid], dst_v)
    _zero_acc(z_h, acc, sid)
    plsc.subcore_barrier()
    _edge_pass(y_h, src_v, dst_v, rows, acc, gsem, ssem, gather=True)
    plsc.subcore_barrier()
    _writeout(acc, o_h, cid, sid)

  return k(y, src, dst, zblk)


def _sc_count(ones_rows, dst, zblk):
  @functools.partial(
      pl.kernel,
      out_type=jax.ShapeDtypeStruct((NCORE, N, 8), jnp.float32),
      mesh=_sc_mesh(),
      compiler_params=_SC_PARAMS,
      scratch_types=[
          pltpu.VMEM((NBLK, EB), jnp.int32),
          pltpu.VMEM((NBUF, EB, 8), jnp.float32),
          pltpu.VMEM_SHARED((N, 8), jnp.float32),
          pltpu.SemaphoreType.DMA((NBUF,)),
          pltpu.SemaphoreType.DMA((NBUF,)),
      ],
  )
  def k(ones_h, dst_h, z_h, o_h, dst_v, rows, acc, gsem, ssem):
    cid = lax.axis_index("c")
    sid = lax.axis_index("s")
    wid = cid * NSUB + sid
    pltpu.sync_copy(dst_h.at[wid], dst_v)
    for b in range(NBUF):
      pltpu.sync_copy(ones_h, rows.at[b])
    _zero_acc(z_h, acc, sid)
    plsc.subcore_barrier()
    _edge_pass(None, None, dst_v, rows, acc, gsem, ssem, gather=False)
    plsc.subcore_barrier()
    _writeout(acc, o_h, cid, sid)

  return k(ones_rows, dst, zblk)


_DOT = functools.partial(
    lax.dot_general,
    dimension_numbers=(((1,), (1,)), ((), ())),
    preferred_element_type=jnp.float32,
)

_W_SPEC = pl.BlockSpec((F, F), lambda i: (0, 0))
_B_SPEC = pl.BlockSpec((8, F), lambda i: (0, 0))
_H_SPEC = pl.BlockSpec((RB, F), lambda i: (i, 0))
_C_SPEC = pl.BlockSpec((RB, FC), lambda i: (i, 0))
_P_SPEC = pl.BlockSpec((NCORE, RB, FC), lambda i: (0, i, 0))
_P8_SPEC = pl.BlockSpec((NCORE, RB, 8), lambda i: (0, i, 0))


def _emit_layer(h, wl_ref, wr_ref, b_ref, c_refs, yr_ref):
  yl = _DOT(h, wl_ref[...])
  for c, c_ref in enumerate(c_refs):
    c_ref[...] = yl[:, c * FC:(c + 1) * FC]
  yr_ref[...] = _DOT(h, wr_ref[...]) + b_ref[0:1, :]


def _tc_first(h0, wl, wr, b):
  def body(h_ref, wl_ref, wr_ref, b_ref, *outs):
    _emit_layer(h_ref[...], wl_ref, wr_ref, b_ref, outs[:NCHUNK], outs[NCHUNK])

  cs = jax.ShapeDtypeStruct((N, FC), jnp.float32)
  return pl.pallas_call(
      body,
      grid=(NRB,),
      in_specs=[_H_SPEC, _W_SPEC, _W_SPEC, _B_SPEC],
      out_specs=[_C_SPEC] * NCHUNK + [_H_SPEC],
      out_shape=[cs] * NCHUNK + [jax.ShapeDtypeStruct((N, F), jnp.float32)],
  )(h0, wl, wr, b)


def _combine(p_refs, cnt_ref, yr_ref):
  seg = jnp.concatenate([p[0] + p[1] for p in p_refs], axis=1)
  cnt = cnt_ref[0, :, 0:1] + cnt_ref[1, :, 0:1]
  inv = 1.0 / jnp.maximum(cnt, 1.0)
  return jax.nn.relu(seg * inv + yr_ref[...])


def _tc_mid(ps, cntp, yr_prev, wl, wr, b):
  def body(*refs):
    p_refs = refs[:NCHUNK]
    cnt_ref, yrp_ref, wl_ref, wr_ref, b_ref = refs[NCHUNK:NCHUNK + 5]
    outs = refs[NCHUNK + 5:]
    h = _combine(p_refs, cnt_ref, yrp_ref)
    _emit_layer(h, wl_ref, wr_ref, b_ref, outs[:NCHUNK], outs[NCHUNK])

  cs = jax.ShapeDtypeStruct((N, FC), jnp.float32)
  return pl.pallas_call(
      body,
      grid=(NRB,),
      in_specs=[_P_SPEC] * NCHUNK + [_P8_SPEC, _H_SPEC, _W_SPEC, _W_SPEC,
                                     _B_SPEC],
      out_specs=[_C_SPEC] * NCHUNK + [_H_SPEC],
      out_shape=[cs] * NCHUNK + [jax.ShapeDtypeStruct((N, F), jnp.float32)],
  )(*ps, cntp, yr_prev, wl, wr, b)


def _tc_last_pre(ps, cntp, yr_prev, w3p, b3):
  def body(*refs):
    p_refs = refs[:NCHUNK]
    cnt_ref, yrp_ref, w_ref, b_ref = refs[NCHUNK:NCHUNK + 4]
    y3_ref, yr3_ref = refs[NCHUNK + 4:]
    h = _combine(p_refs, cnt_ref, yrp_ref)
    y3 = _DOT(h, w_ref[...])
    y3_ref[...] = y3
    yr3_ref[...] = y3[:, 1:2] + b_ref[0, 0]

  return pl.pallas_call(
      body,
      grid=(NRB,),
      in_specs=[_P_SPEC] * NCHUNK + [
          _P8_SPEC, _H_SPEC,
          pl.BlockSpec((8, F), lambda i: (0, 0)),
          pl.BlockSpec(memory_space=pltpu.SMEM)],
      out_specs=[pl.BlockSpec((RB, 8), lambda i: (i, 0)),
                 pl.BlockSpec((RB, 1), lambda i: (i, 0))],
      out_shape=[jax.ShapeDtypeStruct((N, 8), jnp.float32),
                 jax.ShapeDtypeStruct((N, 1), jnp.float32)],
  )(*ps, cntp, yr_prev, w3p, b3)


def _tc_final(p3, cntp, yr3):
  def body(p3_ref, cnt_ref, yr3_ref, out_ref, util_ref, acc_ref):
    i = pl.program_id(0)

    @pl.when(i == 0)
    def _():
      acc_ref[0] = 0.0

    seg = p3_ref[0, :, 0:1] + p3_ref[1, :, 0:1]
    cnt = cnt_ref[0, :, 0:1] + cnt_ref[1, :, 0:1]
    inv = 1.0 / jnp.maximum(cnt, 1.0)
    o = jax.nn.sigmoid(seg * inv + yr3_ref[...])
    out_ref[...] = o
    acc_ref[0] = acc_ref[0] + jnp.sum(o)
    util_ref[0] = acc_ref[0] * (1.0 / N)

  return pl.pallas_call(
      body,
      grid=(NRB,),
      in_specs=[_P8_SPEC, _P8_SPEC, pl.BlockSpec((RB, 1), lambda i: (i, 0))],
      out_specs=[pl.BlockSpec((RB, 1), lambda i: (i, 0)),
                 pl.BlockSpec(memory_space=pltpu.SMEM)],
      out_shape=[jax.ShapeDtypeStruct((N, 1), jnp.float32),
                 jax.ShapeDtypeStruct((1,), jnp.float32)],
      scratch_shapes=[pltpu.SMEM((1,), jnp.float32)],
  )(p3, cntp, yr3)


def kernel(x, diff, rec_prev, hidden_prev, edge_index,
           Wl0, Wr0, b0, Wl1, Wr1, b1, Wl2, Wr2, b2, Wl3, Wr3, b3):
  h0 = jnp.concatenate([x, diff, rec_prev, hidden_prev], axis=1)
  src = edge_index[0].reshape(NW, NBLK, EB)
  dst = edge_index[1].reshape(NW, NBLK, EB)

  zc = jnp.zeros((ZBLK, FC), jnp.float32)
  z8 = jnp.zeros((ZBLK, 8), jnp.float32)
  ones8 = jnp.ones((EB, 8), jnp.float32)

  def pad_b(b):
    return jnp.broadcast_to(b[None, :], (8, F))

  cntp = _sc_count(ones8, dst, z8)

  *cs, yr = _tc_first(h0, Wl0, Wr0, pad_b(b0))
  ps = _sc_segsum_wide(cs, src, dst, zc)

  *cs, yr = _tc_mid(ps, cntp, yr, Wl1, Wr1, pad_b(b1))
  ps = _sc_segsum_wide(cs, src, dst, zc)

  *cs, yr = _tc_mid(ps, cntp, yr, Wl2, Wr2, pad_b(b2))
  ps = _sc_segsum_wide(cs, src, dst, zc)

  w3p = jnp.concatenate([Wl3, Wr3, jnp.zeros((6, F), jnp.float32)], axis=0)
  y3p, yr3 = _tc_last_pre(ps, cntp, yr, w3p, b3.reshape(1, 1))
  p3 = _sc_segsum_narrow(y3p, src, dst, z8)

  out, util = _tc_final(p3, cntp, yr3)
  return out, util[0]


# speedup vs baseline: 7.4297x; 1.3052x over previous
import functools

import jax
import jax.numpy as jnp
from jax import lax
from jax.experimental import pallas as pl
from jax.experimental.pallas import tpu as pltpu
from jax.experimental.pallas import tpu_sc as plsc

N = 50000
E = 800000
F = 96
FP = 128

NCORE = 2
NSUB = 16
NW = NCORE * NSUB
EPW = E // NW
EB = 125
NBLK = EPW // EB
NBUF = 4
ZBLK = 1000
NZB = N // ZBLK
ZPT = -(-NZB // NSUB)

FC = 16
NCHUNK = F // FC

RB = 1000
NRB = N // RB


def _sc_mesh():
  return plsc.VectorSubcoreMesh(core_axis_name="c", subcore_axis_name="s")


_SC_PARAMS = pltpu.CompilerParams(use_tc_tiling_on_sc=False)


def _edge_pass(table_h, src_v, dst_v, rows, acc, gsem, ssem, gather):
  def group(g, carry):
    if gather:
      for b in range(NBUF):
        j = g * NBUF + b

        @pl.when(g > 0)
        def _():
          pltpu.make_async_copy(rows.at[b], acc.at[dst_v.at[j]],
                                ssem.at[b]).wait()

        pltpu.async_copy(table_h.at[src_v.at[j]], rows.at[b], gsem.at[b])
    for b in range(NBUF):
      j = g * NBUF + b
      if gather:
        pltpu.make_async_copy(table_h.at[src_v.at[j]], rows.at[b],
                              gsem.at[b]).wait()
      else:
        @pl.when(g > 0)
        def _():
          pltpu.make_async_copy(rows.at[b], acc.at[dst_v.at[j]],
                                ssem.at[b]).wait()
      pltpu.async_copy(rows.at[b], acc.at[dst_v.at[j]], ssem.at[b], add=True)
    return carry

  lax.fori_loop(0, NBLK // NBUF, group, 0)
  for b in range(NBUF):
    j = NBLK - NBUF + b
    pltpu.make_async_copy(rows.at[b], acc.at[dst_v.at[j]], ssem.at[b]).wait()


def _zero_acc(z_h, acc, sid):
  for t in range(ZPT):
    blk = sid + NSUB * t

    @pl.when(blk < NZB)
    def _():
      pltpu.sync_copy(z_h, acc.at[pl.ds(blk * ZBLK, ZBLK)])


def _writeout(acc, out_h, cid, sid, col):
  w = acc.shape[-1]
  for t in range(ZPT):
    blk = sid + NSUB * t

    @pl.when(blk < NZB)
    def _():
      pltpu.sync_copy(acc.at[pl.ds(blk * ZBLK, ZBLK)],
                      out_h.at[cid, pl.ds(blk * ZBLK, ZBLK), pl.ds(col, w)])


DB = 500
NDB = N // DB
DPT = -(-NDB // NW)


def _sc_deint(y, nchunk):
  @functools.partial(
      pl.kernel,
      out_type=jax.ShapeDtypeStruct((nchunk, N, FC), jnp.float32),
      mesh=_sc_mesh(),
      compiler_params=_SC_PARAMS,
      scratch_types=[
          pltpu.VMEM((DB, FP), jnp.float32),
      ],
  )
  def k(y_h, o_h, buf):
    cid = lax.axis_index("c")
    sid = lax.axis_index("s")
    wid = cid * NSUB + sid
    for t in range(DPT):
      blk = wid + NW * t

      @pl.when(blk < NDB)
      def _():
        pltpu.sync_copy(y_h.at[pl.ds(blk * DB, DB)], buf)
        for c in range(nchunk):
          pltpu.sync_copy(buf.at[:, pl.ds(c * FC, FC)],
                          o_h.at[c, pl.ds(blk * DB, DB)])

  return k(y)


def _sc_segsum_wide(y, src, dst, zblk):
  @functools.partial(
      pl.kernel,
      out_type=jax.ShapeDtypeStruct((NCORE, N, FP), jnp.float32),
      mesh=_sc_mesh(),
      compiler_params=_SC_PARAMS,
      scratch_types=[
          pltpu.VMEM((NBLK, EB), jnp.int32),
          pltpu.VMEM((NBLK, EB), jnp.int32),
          pltpu.VMEM((NBUF, EB, FC), jnp.float32),
          pltpu.VMEM_SHARED((N, FC), jnp.float32),
          pltpu.SemaphoreType.DMA((NBUF,)),
          pltpu.SemaphoreType.DMA((NBUF,)),
      ],
  )
  def k(y_h, src_h, dst_h, z_h, o_h, src_v, dst_v, rows, acc, gsem, ssem):
    cid = lax.axis_index("c")
    sid = lax.axis_index("s")
    wid = cid * NSUB + sid
    pltpu.sync_copy(src_h.at[wid], src_v)
    pltpu.sync_copy(dst_h.at[wid], dst_v)
    for c in range(NCHUNK):
      table = y_h.at[c]
      _zero_acc(z_h, acc, sid)
      plsc.subcore_barrier()
      _edge_pass(table, src_v, dst_v, rows, acc, gsem, ssem, gather=True)
      plsc.subcore_barrier()
      _writeout(acc, o_h, cid, sid, c * FC)
      plsc.subcore_barrier()

  return k(y, src, dst, zblk)


def _sc_segsum_narrow(y, src, dst, zblk):
  @functools.partial(
      pl.kernel,
      out_type=jax.ShapeDtypeStruct((NCORE, N, FP), jnp.float32),
      mesh=_sc_mesh(),
      compiler_params=_SC_PARAMS,
      scratch_types=[
          pltpu.VMEM((NBLK, EB), jnp.int32),
          pltpu.VMEM((NBLK, EB), jnp.int32),
          pltpu.VMEM((NBUF, EB, FC), jnp.float32),
          pltpu.VMEM_SHARED((N, FC), jnp.float32),
          pltpu.SemaphoreType.DMA((NBUF,)),
          pltpu.SemaphoreType.DMA((NBUF,)),
      ],
  )
  def k(y_h, src_h, dst_h, z_h, o_h, src_v, dst_v, rows, acc, gsem, ssem):
    cid = lax.axis_index("c")
    sid = lax.axis_index("s")
    wid = cid * NSUB + sid
    pltpu.sync_copy(src_h.at[wid], src_v)
    pltpu.sync_copy(dst_h.at[wid], dst_v)
    table = y_h.at[0]
    _zero_acc(z_h, acc, sid)
    plsc.subcore_barrier()
    _edge_pass(table, src_v, dst_v, rows, acc, gsem, ssem, gather=True)
    plsc.subcore_barrier()
    _writeout(acc, o_h, cid, sid, 0)

  return k(y, src, dst, zblk)


def _sc_count(ones_rows, dst, zblk):
  @functools.partial(
      pl.kernel,
      out_type=jax.ShapeDtypeStruct((NCORE, N, FP), jnp.float32),
      mesh=_sc_mesh(),
      compiler_params=_SC_PARAMS,
      scratch_types=[
          pltpu.VMEM((NBLK, EB), jnp.int32),
          pltpu.VMEM((NBUF, EB, 8), jnp.float32),
          pltpu.VMEM_SHARED((N, 8), jnp.float32),
          pltpu.SemaphoreType.DMA((NBUF,)),
          pltpu.SemaphoreType.DMA((NBUF,)),
      ],
  )
  def k(ones_h, dst_h, z_h, o_h, dst_v, rows, acc, gsem, ssem):
    cid = lax.axis_index("c")
    sid = lax.axis_index("s")
    wid = cid * NSUB + sid
    pltpu.sync_copy(dst_h.at[wid], dst_v)
    for b in range(NBUF):
      pltpu.sync_copy(ones_h, rows.at[b])
    _zero_acc(z_h, acc, sid)
    plsc.subcore_barrier()
    _edge_pass(None, None, dst_v, rows, acc, gsem, ssem, gather=False)
    plsc.subcore_barrier()
    _writeout(acc, o_h, cid, sid, 0)

  return k(ones_rows, dst, zblk)


_DOT = functools.partial(
    lax.dot_general,
    dimension_numbers=(((1,), (1,)), ((), ())),
    preferred_element_type=jnp.float32,
)

_WP_SPEC = pl.BlockSpec((FP, F), lambda i: (0, 0))
_W_SPEC = pl.BlockSpec((F, F), lambda i: (0, 0))
_B_SPEC = pl.BlockSpec((8, F), lambda i: (0, 0))
_H_SPEC = pl.BlockSpec((RB, F), lambda i: (i, 0))
_Y_SPEC = pl.BlockSpec((RB, FP), lambda i: (i, 0))
_P_SPEC = pl.BlockSpec((NCORE, RB, FP), lambda i: (0, i, 0))


def _emit_layer(h, wlp_ref, wr_ref, b_ref, y_ref, yr_ref):
  y_ref[...] = _DOT(h, wlp_ref[...])
  yr_ref[...] = _DOT(h, wr_ref[...]) + b_ref[0:1, :]


def _tc_first(h0, wlp, wr, b):
  def body(h_ref, wlp_ref, wr_ref, b_ref, y_ref, yr_ref):
    _emit_layer(h_ref[...], wlp_ref, wr_ref, b_ref, y_ref, yr_ref)

  return pl.pallas_call(
      body,
      grid=(NRB,),
      in_specs=[_H_SPEC, _WP_SPEC, _W_SPEC, _B_SPEC],
      out_specs=[_Y_SPEC, _H_SPEC],
      out_shape=[jax.ShapeDtypeStruct((N, FP), jnp.float32),
                 jax.ShapeDtypeStruct((N, F), jnp.float32)],
  )(h0, wlp, wr, b)


def _combine(p_ref, cnt_ref, yr_ref):
  seg = p_ref[0, :, 0:F] + p_ref[1, :, 0:F]
  cnt = cnt_ref[0, :, 0:1] + cnt_ref[1, :, 0:1]
  inv = 1.0 / jnp.maximum(cnt, 1.0)
  return jax.nn.relu(seg * inv + yr_ref[...])


def _tc_mid(ps, cntp, yr_prev, wlp, wr, b):
  def body(p_ref, cnt_ref, yrp_ref, wlp_ref, wr_ref, b_ref, y_ref, yr_ref):
    h = _combine(p_ref, cnt_ref, yrp_ref)
    _emit_layer(h, wlp_ref, wr_ref, b_ref, y_ref, yr_ref)

  return pl.pallas_call(
      body,
      grid=(NRB,),
      in_specs=[_P_SPEC, _P_SPEC, _H_SPEC, _WP_SPEC, _W_SPEC, _B_SPEC],
      out_specs=[_Y_SPEC, _H_SPEC],
      out_shape=[jax.ShapeDtypeStruct((N, FP), jnp.float32),
                 jax.ShapeDtypeStruct((N, F), jnp.float32)],
  )(ps, cntp, yr_prev, wlp, wr, b)


def _tc_last_pre(ps, cntp, yr_prev, w3p, b3):
  def body(p_ref, cnt_ref, yrp_ref, w_ref, b_ref, y3_ref, yr3_ref):
    h = _combine(p_ref, cnt_ref, yrp_ref)
    y3 = _DOT(h, w_ref[...])
    y3_ref[...] = y3
    yr3_ref[...] = y3[:, 1:2] + b_ref[0, 0]

  return pl.pallas_call(
      body,
      grid=(NRB,),
      in_specs=[_P_SPEC, _P_SPEC, _H_SPEC, _WP_SPEC,
                pl.BlockSpec(memory_space=pltpu.SMEM)],
      out_specs=[_Y_SPEC, pl.BlockSpec((RB, 1), lambda i: (i, 0))],
      out_shape=[jax.ShapeDtypeStruct((N, FP), jnp.float32),
                 jax.ShapeDtypeStruct((N, 1), jnp.float32)],
  )(ps, cntp, yr_prev, w3p, b3)


def _tc_final(p3, cntp, yr3):
  def body(p3_ref, cnt_ref, yr3_ref, out_ref, util_ref, acc_ref):
    i = pl.program_id(0)

    @pl.when(i == 0)
    def _():
      acc_ref[0] = 0.0

    seg = p3_ref[0, :, 0:1] + p3_ref[1, :, 0:1]
    cnt = cnt_ref[0, :, 0:1] + cnt_ref[1, :, 0:1]
    inv = 1.0 / jnp.maximum(cnt, 1.0)
    o = jax.nn.sigmoid(seg * inv + yr3_ref[...])
    out_ref[...] = o
    acc_ref[0] = acc_ref[0] + jnp.sum(o)
    util_ref[0] = acc_ref[0] * (1.0 / N)

  return pl.pallas_call(
      body,
      grid=(NRB,),
      in_specs=[_P_SPEC, _P_SPEC, pl.BlockSpec((RB, 1), lambda i: (i, 0))],
      out_specs=[pl.BlockSpec((RB, 1), lambda i: (i, 0)),
                 pl.BlockSpec(memory_space=pltpu.SMEM)],
      out_shape=[jax.ShapeDtypeStruct((N, 1), jnp.float32),
                 jax.ShapeDtypeStruct((1,), jnp.float32)],
      scratch_shapes=[pltpu.SMEM((1,), jnp.float32)],
  )(p3, cntp, yr3)


def kernel(x, diff, rec_prev, hidden_prev, edge_index,
           Wl0, Wr0, b0, Wl1, Wr1, b1, Wl2, Wr2, b2, Wl3, Wr3, b3):
  h0 = jnp.concatenate([x, diff, rec_prev, hidden_prev], axis=1)
  src = edge_index[0].reshape(NW, NBLK, EB)
  dst = edge_index[1].reshape(NW, NBLK, EB)

  zc = jnp.zeros((ZBLK, FC), jnp.float32)
  z8 = jnp.zeros((ZBLK, 8), jnp.float32)
  ones8 = jnp.ones((EB, 8), jnp.float32)

  def pad_wl(wl):
    return jnp.concatenate([wl, jnp.zeros((FP - F, F), jnp.float32)], axis=0)

  def pad_b(b):
    return jnp.broadcast_to(b[None, :], (8, F))

  cntp = _sc_count(ones8, dst, z8)

  y, yr = _tc_first(h0, pad_wl(Wl0), Wr0, pad_b(b0))
  ps = _sc_segsum_wide(_sc_deint(y, NCHUNK), src, dst, zc)

  y, yr = _tc_mid(ps, cntp, yr, pad_wl(Wl1), Wr1, pad_b(b1))
  ps = _sc_segsum_wide(_sc_deint(y, NCHUNK), src, dst, zc)

  y, yr = _tc_mid(ps, cntp, yr, pad_wl(Wl2), Wr2, pad_b(b2))
  ps = _sc_segsum_wide(_sc_deint(y, NCHUNK), src, dst, zc)

  w3p = jnp.concatenate([Wl3, Wr3, jnp.zeros((FP - 2, F), jnp.float32)],
                        axis=0)
  y3p, yr3 = _tc_last_pre(ps, cntp, yr, w3p, b3.reshape(1, 1))
  p3 = _sc_segsum_narrow(_sc_deint(y3p, 1), src, dst, zc)

  out, util = _tc_final(p3, cntp, yr3)
  return out, util[0]


# speedup vs baseline: 8.8658x; 1.1933x over previous
import functools

import jax
import jax.numpy as jnp
from jax import lax
from jax.experimental import pallas as pl
from jax.experimental.pallas import tpu as pltpu
from jax.experimental.pallas import tpu_sc as plsc

N = 50000
E = 800000
F = 96
FP = 128

NCORE = 2
NSUB = 16
NW = NCORE * NSUB
EPW = E // NW
EB = 125
NBLK = EPW // EB
NBUF = 8
ZBLK = 1000
NZB = N // ZBLK
ZPT = -(-NZB // NSUB)

FC = 16
NCHUNK = F // FC

RB = 1000
NRB = N // RB


def _sc_mesh():
  return plsc.VectorSubcoreMesh(core_axis_name="c", subcore_axis_name="s")


_SC_PARAMS = pltpu.CompilerParams(use_tc_tiling_on_sc=False)


def _edge_pass(table_h, src_v, dst_v, rows, acc, gsem, ssem, gather):
  def group(g, carry):
    if gather:
      for b in range(NBUF):
        j = g * NBUF + b

        @pl.when(g > 0)
        def _():
          pltpu.make_async_copy(rows.at[b], acc.at[dst_v.at[j]],
                                ssem.at[b]).wait()

        pltpu.async_copy(table_h.at[src_v.at[j]], rows.at[b], gsem.at[b])
    for b in range(NBUF):
      j = g * NBUF + b
      if gather:
        pltpu.make_async_copy(table_h.at[src_v.at[j]], rows.at[b],
                              gsem.at[b]).wait()
      else:
        @pl.when(g > 0)
        def _():
          pltpu.make_async_copy(rows.at[b], acc.at[dst_v.at[j]],
                                ssem.at[b]).wait()
      pltpu.async_copy(rows.at[b], acc.at[dst_v.at[j]], ssem.at[b], add=True)
    return carry

  lax.fori_loop(0, NBLK // NBUF, group, 0)
  for b in range(NBUF):
    j = NBLK - NBUF + b
    pltpu.make_async_copy(rows.at[b], acc.at[dst_v.at[j]], ssem.at[b]).wait()


def _zero_acc(z_h, acc, sid):
  for t in range(ZPT):
    blk = sid + NSUB * t

    @pl.when(blk < NZB)
    def _():
      pltpu.sync_copy(z_h, acc.at[pl.ds(blk * ZBLK, ZBLK)])


def _writeout(acc, out_h, cid, sid, col):
  w = acc.shape[-1]
  for t in range(ZPT):
    blk = sid + NSUB * t

    @pl.when(blk < NZB)
    def _():
      pltpu.sync_copy(acc.at[pl.ds(blk * ZBLK, ZBLK)],
                      out_h.at[cid, pl.ds(blk * ZBLK, ZBLK), pl.ds(col, w)])


DB = 500
NDB = N // DB
DPT = -(-NDB // NW)


def _sc_deint(y, nchunk):
  width = FP if nchunk > 1 else FC

  @functools.partial(
      pl.kernel,
      out_type=jax.ShapeDtypeStruct((nchunk, N, FC), jnp.float32),
      mesh=_sc_mesh(),
      compiler_params=_SC_PARAMS,
      scratch_types=[
          pltpu.VMEM((2, DB, width), jnp.float32),
          pltpu.SemaphoreType.DMA((2,)),
          pltpu.SemaphoreType.DMA((2,)),
      ],
  )
  def k(y_h, o_h, buf, isem, osem):
    cid = lax.axis_index("c")
    sid = lax.axis_index("s")
    wid = cid * NSUB + sid

    def src_of(t):
      blk = wid + NW * t
      if nchunk > 1:
        return y_h.at[pl.ds(blk * DB, DB)]
      return y_h.at[pl.ds(blk * DB, DB), pl.ds(0, FC)]

    def blk_of(t):
      return wid + NW * t

    def drain_out(t):
      p = t % 2
      for c in range(nchunk):
        pltpu.make_async_copy(buf.at[p].at[:, pl.ds(c * FC, FC)],
                              o_h.at[c, pl.ds(blk_of(t) * DB, DB)],
                              osem.at[p]).wait()

    @pl.when(blk_of(0) < NDB)
    def _():
      pltpu.async_copy(src_of(0), buf.at[0], isem.at[0])
    for t in range(DPT):
      blk = blk_of(t)
      p = t % 2

      @pl.when(blk < NDB)
      def _():
        pltpu.make_async_copy(src_of(t), buf.at[p], isem.at[p]).wait()
        if t >= 1:
          drain_out(t - 1)
        if t + 1 < DPT:
          @pl.when(blk_of(t + 1) < NDB)
          def _():
            pltpu.async_copy(src_of(t + 1), buf.at[1 - p], isem.at[1 - p])
        for c in range(nchunk):
          pltpu.async_copy(buf.at[p].at[:, pl.ds(c * FC, FC)],
                           o_h.at[c, pl.ds(blk * DB, DB)], osem.at[p])
    for t in range(DPT):
      last = (blk_of(t) < NDB)
      if t + 1 < DPT:
        last = last & (blk_of(t + 1) >= NDB)

      @pl.when(last)
      def _():
        drain_out(t)

  return k(y)


def _sc_segsum_wide(y, src, dst, zblk):
  @functools.partial(
      pl.kernel,
      out_type=jax.ShapeDtypeStruct((NCORE, N, FP), jnp.float32),
      mesh=_sc_mesh(),
      compiler_params=_SC_PARAMS,
      scratch_types=[
          pltpu.VMEM((NBLK, EB), jnp.int32),
          pltpu.VMEM((NBLK, EB), jnp.int32),
          pltpu.VMEM((NBUF, EB, FC), jnp.float32),
          pltpu.VMEM_SHARED((N, FC), jnp.float32),
          pltpu.SemaphoreType.DMA((NBUF,)),
          pltpu.SemaphoreType.DMA((NBUF,)),
      ],
  )
  def k(y_h, src_h, dst_h, z_h, o_h, src_v, dst_v, rows, acc, gsem, ssem):
    cid = lax.axis_index("c")
    sid = lax.axis_index("s")
    wid = cid * NSUB + sid
    pltpu.sync_copy(src_h.at[wid], src_v)
    pltpu.sync_copy(dst_h.at[wid], dst_v)
    for c in range(NCHUNK):
      table = y_h.at[c]
      _zero_acc(z_h, acc, sid)
      plsc.subcore_barrier()
      _edge_pass(table, src_v, dst_v, rows, acc, gsem, ssem, gather=True)
      plsc.subcore_barrier()
      _writeout(acc, o_h, cid, sid, c * FC)
      plsc.subcore_barrier()

  return k(y, src, dst, zblk)


def _sc_segsum_narrow(y, src, dst, zblk):
  @functools.partial(
      pl.kernel,
      out_type=jax.ShapeDtypeStruct((NCORE, N, FP), jnp.float32),
      mesh=_sc_mesh(),
      compiler_params=_SC_PARAMS,
      scratch_types=[
          pltpu.VMEM((NBLK, EB), jnp.int32),
          pltpu.VMEM((NBLK, EB), jnp.int32),
          pltpu.VMEM((NBUF, EB, FC), jnp.float32),
          pltpu.VMEM_SHARED((N, FC), jnp.float32),
          pltpu.SemaphoreType.DMA((NBUF,)),
          pltpu.SemaphoreType.DMA((NBUF,)),
      ],
  )
  def k(y_h, src_h, dst_h, z_h, o_h, src_v, dst_v, rows, acc, gsem, ssem):
    cid = lax.axis_index("c")
    sid = lax.axis_index("s")
    wid = cid * NSUB + sid
    pltpu.sync_copy(src_h.at[wid], src_v)
    pltpu.sync_copy(dst_h.at[wid], dst_v)
    table = y_h.at[0]
    _zero_acc(z_h, acc, sid)
    plsc.subcore_barrier()
    _edge_pass(table, src_v, dst_v, rows, acc, gsem, ssem, gather=True)
    plsc.subcore_barrier()
    _writeout(acc, o_h, cid, sid, 0)

  return k(y, src, dst, zblk)


def _sc_count(ones_rows, dst, zblk):
  @functools.partial(
      pl.kernel,
      out_type=jax.ShapeDtypeStruct((NCORE, N, FP), jnp.float32),
      mesh=_sc_mesh(),
      compiler_params=_SC_PARAMS,
      scratch_types=[
          pltpu.VMEM((NBLK, EB), jnp.int32),
          pltpu.VMEM((NBUF, EB, 8), jnp.float32),
          pltpu.VMEM_SHARED((N, 8), jnp.float32),
          pltpu.SemaphoreType.DMA((NBUF,)),
          pltpu.SemaphoreType.DMA((NBUF,)),
      ],
  )
  def k(ones_h, dst_h, z_h, o_h, dst_v, rows, acc, gsem, ssem):
    cid = lax.axis_index("c")
    sid = lax.axis_index("s")
    wid = cid * NSUB + sid
    pltpu.sync_copy(dst_h.at[wid], dst_v)
    for b in range(NBUF):
      pltpu.sync_copy(ones_h, rows.at[b])
    _zero_acc(z_h, acc, sid)
    plsc.subcore_barrier()
    _edge_pass(None, None, dst_v, rows, acc, gsem, ssem, gather=False)
    plsc.subcore_barrier()
    _writeout(acc, o_h, cid, sid, 0)

  return k(ones_rows, dst, zblk)


_DOT = functools.partial(
    lax.dot_general,
    dimension_numbers=(((1,), (1,)), ((), ())),
    preferred_element_type=jnp.float32,
)

_WP_SPEC = pl.BlockSpec((FP, F), lambda i: (0, 0))
_W_SPEC = pl.BlockSpec((F, F), lambda i: (0, 0))
_B_SPEC = pl.BlockSpec((8, F), lambda i: (0, 0))
_H_SPEC = pl.BlockSpec((RB, F), lambda i: (i, 0))
_Y_SPEC = pl.BlockSpec((RB, FP), lambda i: (i, 0))
_P_SPEC = pl.BlockSpec((NCORE, RB, FP), lambda i: (0, i, 0))


def _emit_layer(h, wlp_ref, wr_ref, b_ref, y_ref, yr_ref):
  y_ref[...] = _DOT(h, wlp_ref[...])
  yr_ref[...] = _DOT(h, wr_ref[...]) + b_ref[0:1, :]


def _tc_first(x, diff, rec_prev, hidden_prev, wlp, wr, b):
  def body(x_ref, d_ref, r_ref, hp_ref, wlp_ref, wr_ref, b_ref, y_ref, yr_ref):
    h = jnp.concatenate(
        [x_ref[...], d_ref[...], r_ref[...], hp_ref[...]], axis=1)
    _emit_layer(h, wlp_ref, wr_ref, b_ref, y_ref, yr_ref)

  return pl.pallas_call(
      body,
      grid=(NRB,),
      in_specs=[pl.BlockSpec((RB, 48), lambda i: (i, 0)),
                pl.BlockSpec((RB, 24), lambda i: (i, 0)),
                pl.BlockSpec((RB, 12), lambda i: (i, 0)),
                pl.BlockSpec((RB, 12), lambda i: (i, 0)),
                _WP_SPEC, _W_SPEC, _B_SPEC],
      out_specs=[_Y_SPEC, _H_SPEC],
      out_shape=[jax.ShapeDtypeStruct((N, FP), jnp.float32),
                 jax.ShapeDtypeStruct((N, F), jnp.float32)],
  )(x, diff, rec_prev, hidden_prev, wlp, wr, b)


def _combine(p_ref, cnt_ref, yr_ref):
  seg = p_ref[0, :, 0:F] + p_ref[1, :, 0:F]
  cnt = cnt_ref[0, :, 0:1] + cnt_ref[1, :, 0:1]
  inv = 1.0 / jnp.maximum(cnt, 1.0)
  return jax.nn.relu(seg * inv + yr_ref[...])


def _tc_mid(ps, cntp, yr_prev, wlp, wr, b):
  def body(p_ref, cnt_ref, yrp_ref, wlp_ref, wr_ref, b_ref, y_ref, yr_ref):
    h = _combine(p_ref, cnt_ref, yrp_ref)
    _emit_layer(h, wlp_ref, wr_ref, b_ref, y_ref, yr_ref)

  return pl.pallas_call(
      body,
      grid=(NRB,),
      in_specs=[_P_SPEC, _P_SPEC, _H_SPEC, _WP_SPEC, _W_SPEC, _B_SPEC],
      out_specs=[_Y_SPEC, _H_SPEC],
      out_shape=[jax.ShapeDtypeStruct((N, FP), jnp.float32),
                 jax.ShapeDtypeStruct((N, F), jnp.float32)],
  )(ps, cntp, yr_prev, wlp, wr, b)


def _tc_last_pre(ps, cntp, yr_prev, w3p, b3):
  def body(p_ref, cnt_ref, yrp_ref, w_ref, b_ref, y3_ref, yr3_ref):
    h = _combine(p_ref, cnt_ref, yrp_ref)
    y3 = _DOT(h, w_ref[...])
    y3_ref[...] = y3
    yr3_ref[...] = y3[:, 1:2] + b_ref[0, 0]

  return pl.pallas_call(
      body,
      grid=(NRB,),
      in_specs=[_P_SPEC, _P_SPEC, _H_SPEC, _WP_SPEC,
                pl.BlockSpec(memory_space=pltpu.SMEM)],
      out_specs=[_Y_SPEC, pl.BlockSpec((RB, 1), lambda i: (i, 0))],
      out_shape=[jax.ShapeDtypeStruct((N, FP), jnp.float32),
                 jax.ShapeDtypeStruct((N, 1), jnp.float32)],
  )(ps, cntp, yr_prev, w3p, b3)


def _tc_final(p3, cntp, yr3):
  def body(p3_ref, cnt_ref, yr3_ref, out_ref, util_ref, acc_ref):
    i = pl.program_id(0)

    @pl.when(i == 0)
    def _():
      acc_ref[0] = 0.0

    seg = p3_ref[0, :, 0:1] + p3_ref[1, :, 0:1]
    cnt = cnt_ref[0, :, 0:1] + cnt_ref[1, :, 0:1]
    inv = 1.0 / jnp.maximum(cnt, 1.0)
    o = jax.nn.sigmoid(seg * inv + yr3_ref[...])
    out_ref[...] = o
    acc_ref[0] = acc_ref[0] + jnp.sum(o)
    util_ref[0] = acc_ref[0] * (1.0 / N)

  return pl.pallas_call(
      body,
      grid=(NRB,),
      in_specs=[_P_SPEC, _P_SPEC, pl.BlockSpec((RB, 1), lambda i: (i, 0))],
      out_specs=[pl.BlockSpec((RB, 1), lambda i: (i, 0)),
                 pl.BlockSpec(memory_space=pltpu.SMEM)],
      out_shape=[jax.ShapeDtypeStruct((N, 1), jnp.float32),
                 jax.ShapeDtypeStruct((1,), jnp.float32)],
      scratch_shapes=[pltpu.SMEM((1,), jnp.float32)],
  )(p3, cntp, yr3)


def kernel(x, diff, rec_prev, hidden_prev, edge_index,
           Wl0, Wr0, b0, Wl1, Wr1, b1, Wl2, Wr2, b2, Wl3, Wr3, b3):
  src = edge_index[0].reshape(NW, NBLK, EB)
  dst = edge_index[1].reshape(NW, NBLK, EB)

  zc = jnp.zeros((ZBLK, FC), jnp.float32)
  z8 = jnp.zeros((ZBLK, 8), jnp.float32)
  ones8 = jnp.ones((EB, 8), jnp.float32)

  def pad_wl(wl):
    return jnp.concatenate([wl, jnp.zeros((FP - F, F), jnp.float32)], axis=0)

  def pad_b(b):
    return jnp.broadcast_to(b[None, :], (8, F))

  cntp = _sc_count(ones8, dst, z8)

  y, yr = _tc_first(x, diff, rec_prev, hidden_prev,
                    pad_wl(Wl0), Wr0, pad_b(b0))
  ps = _sc_segsum_wide(_sc_deint(y, NCHUNK), src, dst, zc)

  y, yr = _tc_mid(ps, cntp, yr, pad_wl(Wl1), Wr1, pad_b(b1))
  ps = _sc_segsum_wide(_sc_deint(y, NCHUNK), src, dst, zc)

  y, yr = _tc_mid(ps, cntp, yr, pad_wl(Wl2), Wr2, pad_b(b2))
  ps = _sc_segsum_wide(_sc_deint(y, NCHUNK), src, dst, zc)

  w3p = jnp.concatenate([Wl3, Wr3, jnp.zeros((FP - 2, F), jnp.float32)],
                        axis=0)
  y3p, yr3 = _tc_last_pre(ps, cntp, yr, w3p, b3.reshape(1, 1))
  p3 = _sc_segsum_narrow(_sc_deint(y3p, 1), src, dst, zc)

  out, util = _tc_final(p3, cntp, yr3)
  return out, util[0]


# speedup vs baseline: 8.9496x; 1.0094x over previous
import functools

import jax
import jax.numpy as jnp
from jax import lax
from jax.experimental import pallas as pl
from jax.experimental.pallas import tpu as pltpu
from jax.experimental.pallas import tpu_sc as plsc

N = 50000
E = 800000
F = 96
FP = 128

NCORE = 2
NSUB = 16
NW = NCORE * NSUB
EPW = E // NW
EB = 125
NBLK = EPW // EB
NBUF = 10
ZBLK = 1000
NZB = N // ZBLK
ZPT = -(-NZB // NSUB)

FC = 16
NCHUNK = F // FC

RB = 1000
NRB = N // RB


def _sc_mesh():
  return plsc.VectorSubcoreMesh(core_axis_name="c", subcore_axis_name="s")


_SC_PARAMS = pltpu.CompilerParams(use_tc_tiling_on_sc=False)


def _edge_pass(table_h, src_v, dst_v, rows, acc, gsem, ssem, gather):
  def group(g, carry):
    if gather:
      for b in range(NBUF):
        j = g * NBUF + b

        @pl.when(g > 0)
        def _():
          pltpu.make_async_copy(rows.at[b], acc.at[dst_v.at[j]],
                                ssem.at[b]).wait()

        pltpu.async_copy(table_h.at[src_v.at[j]], rows.at[b], gsem.at[b])
    for b in range(NBUF):
      j = g * NBUF + b
      if gather:
        pltpu.make_async_copy(table_h.at[src_v.at[j]], rows.at[b],
                              gsem.at[b]).wait()
      else:
        @pl.when(g > 0)
        def _():
          pltpu.make_async_copy(rows.at[b], acc.at[dst_v.at[j]],
                                ssem.at[b]).wait()
      pltpu.async_copy(rows.at[b], acc.at[dst_v.at[j]], ssem.at[b], add=True)
    return carry

  lax.fori_loop(0, NBLK // NBUF, group, 0)
  for b in range(NBUF):
    j = NBLK - NBUF + b
    pltpu.make_async_copy(rows.at[b], acc.at[dst_v.at[j]], ssem.at[b]).wait()


def _zero_acc(z_h, acc, sid):
  for t in range(ZPT):
    blk = sid + NSUB * t

    @pl.when(blk < NZB)
    def _():
      pltpu.sync_copy(z_h, acc.at[pl.ds(blk * ZBLK, ZBLK)])


def _writeout(acc, out_h, cid, sid, col):
  w = acc.shape[-1]
  for t in range(ZPT):
    blk = sid + NSUB * t

    @pl.when(blk < NZB)
    def _():
      pltpu.sync_copy(acc.at[pl.ds(blk * ZBLK, ZBLK)],
                      out_h.at[cid, pl.ds(blk * ZBLK, ZBLK), pl.ds(col, w)])


DB = 500
NDB = N // DB
DPT = -(-NDB // NW)


def _sc_deint(y, nchunk):
  width = FP if nchunk > 1 else FC

  @functools.partial(
      pl.kernel,
      out_type=jax.ShapeDtypeStruct((nchunk, N, FC), jnp.float32),
      mesh=_sc_mesh(),
      compiler_params=_SC_PARAMS,
      scratch_types=[
          pltpu.VMEM((2, DB, width), jnp.float32),
          pltpu.SemaphoreType.DMA((2,)),
          pltpu.SemaphoreType.DMA((2,)),
      ],
  )
  def k(y_h, o_h, buf, isem, osem):
    cid = lax.axis_index("c")
    sid = lax.axis_index("s")
    wid = cid * NSUB + sid

    def src_of(t):
      blk = wid + NW * t
      if nchunk > 1:
        return y_h.at[pl.ds(blk * DB, DB)]
      return y_h.at[pl.ds(blk * DB, DB), pl.ds(0, FC)]

    def blk_of(t):
      return wid + NW * t

    def drain_out(t):
      p = t % 2
      for c in range(nchunk):
        pltpu.make_async_copy(buf.at[p].at[:, pl.ds(c * FC, FC)],
                              o_h.at[c, pl.ds(blk_of(t) * DB, DB)],
                              osem.at[p]).wait()

    @pl.when(blk_of(0) < NDB)
    def _():
      pltpu.async_copy(src_of(0), buf.at[0], isem.at[0])
    for t in range(DPT):
      blk = blk_of(t)
      p = t % 2

      @pl.when(blk < NDB)
      def _():
        pltpu.make_async_copy(src_of(t), buf.at[p], isem.at[p]).wait()
        if t >= 1:
          drain_out(t - 1)
        if t + 1 < DPT:
          @pl.when(blk_of(t + 1) < NDB)
          def _():
            pltpu.async_copy(src_of(t + 1), buf.at[1 - p], isem.at[1 - p])
        for c in range(nchunk):
          pltpu.async_copy(buf.at[p].at[:, pl.ds(c * FC, FC)],
                           o_h.at[c, pl.ds(blk * DB, DB)], osem.at[p])
    for t in range(DPT):
      last = (blk_of(t) < NDB)
      if t + 1 < DPT:
        last = last & (blk_of(t + 1) >= NDB)

      @pl.when(last)
      def _():
        drain_out(t)

  return k(y)


def _sc_segsum_wide(y, eidx, zblk):
  @functools.partial(
      pl.kernel,
      out_type=jax.ShapeDtypeStruct((NCORE, N, FP), jnp.float32),
      mesh=_sc_mesh(),
      compiler_params=_SC_PARAMS,
      scratch_types=[
          pltpu.VMEM((NBLK, EB), jnp.int32),
          pltpu.VMEM((NBLK, EB), jnp.int32),
          pltpu.VMEM((NBUF, EB, FC), jnp.float32),
          pltpu.VMEM_SHARED((N, FC), jnp.float32),
          pltpu.SemaphoreType.DMA((NBUF,)),
          pltpu.SemaphoreType.DMA((NBUF,)),
      ],
  )
  def k(y_h, e_h, z_h, o_h, src_v, dst_v, rows, acc, gsem, ssem):
    cid = lax.axis_index("c")
    sid = lax.axis_index("s")
    wid = cid * NSUB + sid
    pltpu.sync_copy(e_h.at[0, wid], src_v)
    pltpu.sync_copy(e_h.at[1, wid], dst_v)
    for c in range(NCHUNK):
      table = y_h.at[c]
      _zero_acc(z_h, acc, sid)
      plsc.subcore_barrier()
      _edge_pass(table, src_v, dst_v, rows, acc, gsem, ssem, gather=True)
      plsc.subcore_barrier()
      _writeout(acc, o_h, cid, sid, c * FC)
      plsc.subcore_barrier()

  return k(y, eidx, zblk)


def _sc_segsum_narrow(y, eidx, zblk):
  @functools.partial(
      pl.kernel,
      out_type=jax.ShapeDtypeStruct((NCORE, N, FP), jnp.float32),
      mesh=_sc_mesh(),
      compiler_params=_SC_PARAMS,
      scratch_types=[
          pltpu.VMEM((NBLK, EB), jnp.int32),
          pltpu.VMEM((NBLK, EB), jnp.int32),
          pltpu.VMEM((NBUF, EB, FC), jnp.float32),
          pltpu.VMEM_SHARED((N, FC), jnp.float32),
          pltpu.SemaphoreType.DMA((NBUF,)),
          pltpu.SemaphoreType.DMA((NBUF,)),
      ],
  )
  def k(y_h, e_h, z_h, o_h, src_v, dst_v, rows, acc, gsem, ssem):
    cid = lax.axis_index("c")
    sid = lax.axis_index("s")
    wid = cid * NSUB + sid
    pltpu.sync_copy(e_h.at[0, wid], src_v)
    pltpu.sync_copy(e_h.at[1, wid], dst_v)
    table = y_h.at[0]
    _zero_acc(z_h, acc, sid)
    plsc.subcore_barrier()
    _edge_pass(table, src_v, dst_v, rows, acc, gsem, ssem, gather=True)
    plsc.subcore_barrier()
    _writeout(acc, o_h, cid, sid, 0)

  return k(y, eidx, zblk)


def _sc_count(ones_rows, eidx, zblk):
  @functools.partial(
      pl.kernel,
      out_type=jax.ShapeDtypeStruct((NCORE, N, FP), jnp.float32),
      mesh=_sc_mesh(),
      compiler_params=_SC_PARAMS,
      scratch_types=[
          pltpu.VMEM((NBLK, EB), jnp.int32),
          pltpu.VMEM((NBUF, EB, 8), jnp.float32),
          pltpu.VMEM_SHARED((N, 8), jnp.float32),
          pltpu.SemaphoreType.DMA((NBUF,)),
          pltpu.SemaphoreType.DMA((NBUF,)),
      ],
  )
  def k(ones_h, e_h, z_h, o_h, dst_v, rows, acc, gsem, ssem):
    cid = lax.axis_index("c")
    sid = lax.axis_index("s")
    wid = cid * NSUB + sid
    pltpu.sync_copy(e_h.at[1, wid], dst_v)
    for b in range(NBUF):
      pltpu.sync_copy(ones_h, rows.at[b])
    _zero_acc(z_h, acc, sid)
    plsc.subcore_barrier()
    _edge_pass(None, None, dst_v, rows, acc, gsem, ssem, gather=False)
    plsc.subcore_barrier()
    _writeout(acc, o_h, cid, sid, 0)

  return k(ones_rows, eidx, zblk)


_DOT = functools.partial(
    lax.dot_general,
    dimension_numbers=(((1,), (1,)), ((), ())),
    preferred_element_type=jnp.float32,
)

_WP_SPEC = pl.BlockSpec((FP, F), lambda i: (0, 0))
_W_SPEC = pl.BlockSpec((F, F), lambda i: (0, 0))
_B_SPEC = pl.BlockSpec((8, F), lambda i: (0, 0))
_H_SPEC = pl.BlockSpec((RB, F), lambda i: (i, 0))
_Y_SPEC = pl.BlockSpec((RB, FP), lambda i: (i, 0))
_P_SPEC = pl.BlockSpec((NCORE, RB, FP), lambda i: (0, i, 0))


def _emit_layer(h, wlp_ref, wr_ref, b_ref, y_ref, yr_ref):
  y_ref[...] = _DOT(h, wlp_ref[...])
  yr_ref[...] = _DOT(h, wr_ref[...]) + b_ref[0:1, :]


def _tc_first(x, diff, rec_prev, hidden_prev, wlp, wr, b):
  def body(x_ref, d_ref, r_ref, hp_ref, wlp_ref, wr_ref, b_ref, y_ref, yr_ref):
    h = jnp.concatenate(
        [x_ref[...], d_ref[...], r_ref[...], hp_ref[...]], axis=1)
    _emit_layer(h, wlp_ref, wr_ref, b_ref, y_ref, yr_ref)

  return pl.pallas_call(
      body,
      grid=(NRB,),
      in_specs=[pl.BlockSpec((RB, 48), lambda i: (i, 0)),
                pl.BlockSpec((RB, 24), lambda i: (i, 0)),
                pl.BlockSpec((RB, 12), lambda i: (i, 0)),
                pl.BlockSpec((RB, 12), lambda i: (i, 0)),
                _WP_SPEC, _W_SPEC, _B_SPEC],
      out_specs=[_Y_SPEC, _H_SPEC],
      out_shape=[jax.ShapeDtypeStruct((N, FP), jnp.float32),
                 jax.ShapeDtypeStruct((N, F), jnp.float32)],
  )(x, diff, rec_prev, hidden_prev, wlp, wr, b)


def _combine(p_ref, cnt_ref, yr_ref):
  seg = p_ref[0, :, 0:F] + p_ref[1, :, 0:F]
  cnt = cnt_ref[0, :, 0:1] + cnt_ref[1, :, 0:1]
  inv = 1.0 / jnp.maximum(cnt, 1.0)
  return jax.nn.relu(seg * inv + yr_ref[...])


def _tc_mid(ps, cntp, yr_prev, wlp, wr, b):
  def body(p_ref, cnt_ref, yrp_ref, wlp_ref, wr_ref, b_ref, y_ref, yr_ref):
    h = _combine(p_ref, cnt_ref, yrp_ref)
    _emit_layer(h, wlp_ref, wr_ref, b_ref, y_ref, yr_ref)

  return pl.pallas_call(
      body,
      grid=(NRB,),
      in_specs=[_P_SPEC, _P_SPEC, _H_SPEC, _WP_SPEC, _W_SPEC, _B_SPEC],
      out_specs=[_Y_SPEC, _H_SPEC],
      out_shape=[jax.ShapeDtypeStruct((N, FP), jnp.float32),
                 jax.ShapeDtypeStruct((N, F), jnp.float32)],
  )(ps, cntp, yr_prev, wlp, wr, b)


def _tc_last_pre(ps, cntp, yr_prev, w3p, b3):
  def body(p_ref, cnt_ref, yrp_ref, w_ref, b_ref, y3_ref, yr3_ref):
    h = _combine(p_ref, cnt_ref, yrp_ref)
    y3 = _DOT(h, w_ref[...])
    y3_ref[...] = y3
    yr3_ref[...] = y3[:, 1:2] + b_ref[0, 0]

  return pl.pallas_call(
      body,
      grid=(NRB,),
      in_specs=[_P_SPEC, _P_SPEC, _H_SPEC, _WP_SPEC,
                pl.BlockSpec(memory_space=pltpu.SMEM)],
      out_specs=[_Y_SPEC, pl.BlockSpec((RB, 1), lambda i: (i, 0))],
      out_shape=[jax.ShapeDtypeStruct((N, FP), jnp.float32),
                 jax.ShapeDtypeStruct((N, 1), jnp.float32)],
  )(ps, cntp, yr_prev, w3p, b3)


def _tc_final(p3, cntp, yr3):
  def body(p3_ref, cnt_ref, yr3_ref, out_ref, util_ref, acc_ref):
    i = pl.program_id(0)

    @pl.when(i == 0)
    def _():
      acc_ref[0] = 0.0

    seg = p3_ref[0, :, 0:1] + p3_ref[1, :, 0:1]
    cnt = cnt_ref[0, :, 0:1] + cnt_ref[1, :, 0:1]
    inv = 1.0 / jnp.maximum(cnt, 1.0)
    o = jax.nn.sigmoid(seg * inv + yr3_ref[...])
    out_ref[...] = o
    acc_ref[0] = acc_ref[0] + jnp.sum(o)
    util_ref[0] = acc_ref[0] * (1.0 / N)

  return pl.pallas_call(
      body,
      grid=(NRB,),
      in_specs=[_P_SPEC, _P_SPEC, pl.BlockSpec((RB, 1), lambda i: (i, 0))],
      out_specs=[pl.BlockSpec((RB, 1), lambda i: (i, 0)),
                 pl.BlockSpec(memory_space=pltpu.SMEM)],
      out_shape=[jax.ShapeDtypeStruct((N, 1), jnp.float32),
                 jax.ShapeDtypeStruct((1,), jnp.float32)],
      scratch_shapes=[pltpu.SMEM((1,), jnp.float32)],
  )(p3, cntp, yr3)


def kernel(x, diff, rec_prev, hidden_prev, edge_index,
           Wl0, Wr0, b0, Wl1, Wr1, b1, Wl2, Wr2, b2, Wl3, Wr3, b3):
  eidx = edge_index.reshape(2, NW, NBLK, EB)

  zc = jnp.zeros((ZBLK, FC), jnp.float32)
  z8 = jnp.zeros((ZBLK, 8), jnp.float32)
  ones8 = jnp.ones((EB, 8), jnp.float32)

  def pad_wl(wl):
    return jnp.concatenate([wl, jnp.zeros((FP - F, F), jnp.float32)], axis=0)

  def pad_b(b):
    return jnp.broadcast_to(b[None, :], (8, F))

  cntp = _sc_count(ones8, eidx, z8)

  y, yr = _tc_first(x, diff, rec_prev, hidden_prev,
                    pad_wl(Wl0), Wr0, pad_b(b0))
  y, cntp = lax.optimization_barrier((y, cntp))
  ps = _sc_segsum_wide(_sc_deint(y, NCHUNK), eidx, zc)

  y, yr = _tc_mid(ps, cntp, yr, pad_wl(Wl1), Wr1, pad_b(b1))
  ps = _sc_segsum_wide(_sc_deint(y, NCHUNK), eidx, zc)

  y, yr = _tc_mid(ps, cntp, yr, pad_wl(Wl2), Wr2, pad_b(b2))
  ps = _sc_segsum_wide(_sc_deint(y, NCHUNK), eidx, zc)

  w3p = jnp.concatenate([Wl3, Wr3, jnp.zeros((FP - 2, F), jnp.float32)],
                        axis=0)
  y3p, yr3 = _tc_last_pre(ps, cntp, yr, w3p, b3.reshape(1, 1))
  p3 = _sc_segsum_narrow(_sc_deint(y3p, 1), eidx, zc)

  out, util = _tc_final(p3, cntp, yr3)
  return out, util[0]


# speedup vs baseline: 9.1762x; 1.0253x over previous
import functools

import jax
import jax.numpy as jnp
from jax import lax
from jax.experimental import pallas as pl
from jax.experimental.pallas import tpu as pltpu
from jax.experimental.pallas import tpu_sc as plsc

N = 50000
E = 800000
F = 96
FP = 128

NCORE = 2
NSUB = 16
NW = NCORE * NSUB
EPW = E // NW
EB = 125
NBLK = EPW // EB
NBUF = 10
ZBLK = 1000
NZB = N // ZBLK
ZPT = -(-NZB // NSUB)

FC = 16
NCHUNK = F // FC

RB = 2000
NRB = N // RB


def _sc_mesh():
  return plsc.VectorSubcoreMesh(core_axis_name="c", subcore_axis_name="s")


_SC_PARAMS = pltpu.CompilerParams(use_tc_tiling_on_sc=False)


def _edge_pass(table_h, src_v, dst_v, rows, acc, gsem, ssem, gather):
  def group(g, carry):
    if gather:
      for b in range(NBUF):
        j = g * NBUF + b

        @pl.when(g > 0)
        def _():
          pltpu.make_async_copy(rows.at[b], acc.at[dst_v.at[j]],
                                ssem.at[b]).wait()

        pltpu.async_copy(table_h.at[src_v.at[j]], rows.at[b], gsem.at[b])
    for b in range(NBUF):
      j = g * NBUF + b
      if gather:
        pltpu.make_async_copy(table_h.at[src_v.at[j]], rows.at[b],
                              gsem.at[b]).wait()
      else:
        @pl.when(g > 0)
        def _():
          pltpu.make_async_copy(rows.at[b], acc.at[dst_v.at[j]],
                                ssem.at[b]).wait()
      pltpu.async_copy(rows.at[b], acc.at[dst_v.at[j]], ssem.at[b], add=True)
    return carry

  lax.fori_loop(0, NBLK // NBUF, group, 0)
  for b in range(NBUF):
    j = NBLK - NBUF + b
    pltpu.make_async_copy(rows.at[b], acc.at[dst_v.at[j]], ssem.at[b]).wait()


def _zero_acc(z_h, acc, sid):
  for t in range(ZPT):
    blk = sid + NSUB * t

    @pl.when(blk < NZB)
    def _():
      pltpu.sync_copy(z_h, acc.at[pl.ds(blk * ZBLK, ZBLK)])


def _writeout(acc, out_h, cid, sid, col):
  w = acc.shape[-1]
  for t in range(ZPT):
    blk = sid + NSUB * t

    @pl.when(blk < NZB)
    def _():
      pltpu.sync_copy(acc.at[pl.ds(blk * ZBLK, ZBLK)],
                      out_h.at[cid, pl.ds(blk * ZBLK, ZBLK), pl.ds(col, w)])


DB = 500
NDB = N // DB
DPT = -(-NDB // NW)


def _sc_deint(y, nchunk):
  width = FP if nchunk > 1 else FC

  @functools.partial(
      pl.kernel,
      out_type=jax.ShapeDtypeStruct((nchunk, N, FC), jnp.float32),
      mesh=_sc_mesh(),
      compiler_params=_SC_PARAMS,
      scratch_types=[
          pltpu.VMEM((2, DB, width), jnp.float32),
          pltpu.SemaphoreType.DMA((2,)),
          pltpu.SemaphoreType.DMA((2,)),
      ],
  )
  def k(y_h, o_h, buf, isem, osem):
    cid = lax.axis_index("c")
    sid = lax.axis_index("s")
    wid = cid * NSUB + sid

    def src_of(t):
      blk = wid + NW * t
      if nchunk > 1:
        return y_h.at[pl.ds(blk * DB, DB)]
      return y_h.at[pl.ds(blk * DB, DB), pl.ds(0, FC)]

    def blk_of(t):
      return wid + NW * t

    def drain_out(t):
      p = t % 2
      for c in range(nchunk):
        pltpu.make_async_copy(buf.at[p].at[:, pl.ds(c * FC, FC)],
                              o_h.at[c, pl.ds(blk_of(t) * DB, DB)],
                              osem.at[p]).wait()

    @pl.when(blk_of(0) < NDB)
    def _():
      pltpu.async_copy(src_of(0), buf.at[0], isem.at[0])
    for t in range(DPT):
      blk = blk_of(t)
      p = t % 2

      @pl.when(blk < NDB)
      def _():
        pltpu.make_async_copy(src_of(t), buf.at[p], isem.at[p]).wait()
        if t >= 1:
          drain_out(t - 1)
        if t + 1 < DPT:
          @pl.when(blk_of(t + 1) < NDB)
          def _():
            pltpu.async_copy(src_of(t + 1), buf.at[1 - p], isem.at[1 - p])
        for c in range(nchunk):
          pltpu.async_copy(buf.at[p].at[:, pl.ds(c * FC, FC)],
                           o_h.at[c, pl.ds(blk * DB, DB)], osem.at[p])
    for t in range(DPT):
      last = (blk_of(t) < NDB)
      if t + 1 < DPT:
        last = last & (blk_of(t + 1) >= NDB)

      @pl.when(last)
      def _():
        drain_out(t)

  return k(y)


def _sc_segsum_wide(y, eidx, zblk):
  @functools.partial(
      pl.kernel,
      out_type=jax.ShapeDtypeStruct((NCORE, N, FP), jnp.float32),
      mesh=_sc_mesh(),
      compiler_params=_SC_PARAMS,
      scratch_types=[
          pltpu.VMEM((NBLK, EB), jnp.int32),
          pltpu.VMEM((NBLK, EB), jnp.int32),
          pltpu.VMEM((NBUF, EB, FC), jnp.float32),
          pltpu.VMEM_SHARED((N, FC), jnp.float32),
          pltpu.SemaphoreType.DMA((NBUF,)),
          pltpu.SemaphoreType.DMA((NBUF,)),
      ],
  )
  def k(y_h, e_h, z_h, o_h, src_v, dst_v, rows, acc, gsem, ssem):
    cid = lax.axis_index("c")
    sid = lax.axis_index("s")
    wid = cid * NSUB + sid
    pltpu.sync_copy(e_h.at[0, wid], src_v)
    pltpu.sync_copy(e_h.at[1, wid], dst_v)
    for c in range(NCHUNK):
      table = y_h.at[c]
      _zero_acc(z_h, acc, sid)
      plsc.subcore_barrier()
      _edge_pass(table, src_v, dst_v, rows, acc, gsem, ssem, gather=True)
      plsc.subcore_barrier()
      _writeout(acc, o_h, cid, sid, c * FC)
      plsc.subcore_barrier()

  return k(y, eidx, zblk)


def _sc_segsum_narrow(y, eidx, zblk):
  @functools.partial(
      pl.kernel,
      out_type=jax.ShapeDtypeStruct((NCORE, N, FP), jnp.float32),
      mesh=_sc_mesh(),
      compiler_params=_SC_PARAMS,
      scratch_types=[
          pltpu.VMEM((NBLK, EB), jnp.int32),
          pltpu.VMEM((NBLK, EB), jnp.int32),
          pltpu.VMEM((NBUF, EB, FC), jnp.float32),
          pltpu.VMEM_SHARED((N, FC), jnp.float32),
          pltpu.SemaphoreType.DMA((NBUF,)),
          pltpu.SemaphoreType.DMA((NBUF,)),
      ],
  )
  def k(y_h, e_h, z_h, o_h, src_v, dst_v, rows, acc, gsem, ssem):
    cid = lax.axis_index("c")
    sid = lax.axis_index("s")
    wid = cid * NSUB + sid
    pltpu.sync_copy(e_h.at[0, wid], src_v)
    pltpu.sync_copy(e_h.at[1, wid], dst_v)
    table = y_h.at[0]
    _zero_acc(z_h, acc, sid)
    plsc.subcore_barrier()
    _edge_pass(table, src_v, dst_v, rows, acc, gsem, ssem, gather=True)
    plsc.subcore_barrier()
    _writeout(acc, o_h, cid, sid, 0)

  return k(y, eidx, zblk)


def _sc_count(ones_rows, eidx, zblk):
  @functools.partial(
      pl.kernel,
      out_type=jax.ShapeDtypeStruct((NCORE, N, FP), jnp.float32),
      mesh=_sc_mesh(),
      compiler_params=_SC_PARAMS,
      scratch_types=[
          pltpu.VMEM((NBLK, EB), jnp.int32),
          pltpu.VMEM((NBUF, EB, 8), jnp.float32),
          pltpu.VMEM_SHARED((N, 8), jnp.float32),
          pltpu.SemaphoreType.DMA((NBUF,)),
          pltpu.SemaphoreType.DMA((NBUF,)),
      ],
  )
  def k(ones_h, e_h, z_h, o_h, dst_v, rows, acc, gsem, ssem):
    cid = lax.axis_index("c")
    sid = lax.axis_index("s")
    wid = cid * NSUB + sid
    pltpu.sync_copy(e_h.at[1, wid], dst_v)
    for b in range(NBUF):
      pltpu.sync_copy(ones_h, rows.at[b])
    _zero_acc(z_h, acc, sid)
    plsc.subcore_barrier()
    _edge_pass(None, None, dst_v, rows, acc, gsem, ssem, gather=False)
    plsc.subcore_barrier()
    _writeout(acc, o_h, cid, sid, 0)

  return k(ones_rows, eidx, zblk)


_DOT = functools.partial(
    lax.dot_general,
    dimension_numbers=(((1,), (1,)), ((), ())),
    preferred_element_type=jnp.float32,
)

_WP_SPEC = pl.BlockSpec((FP, F), lambda i: (0, 0))
_W_SPEC = pl.BlockSpec((F, F), lambda i: (0, 0))
_B_SPEC = pl.BlockSpec((8, F), lambda i: (0, 0))
_H_SPEC = pl.BlockSpec((RB, F), lambda i: (i, 0))
_Y_SPEC = pl.BlockSpec((RB, FP), lambda i: (i, 0))
_P_SPEC = pl.BlockSpec((NCORE, RB, FP), lambda i: (0, i, 0))


def _emit_layer(h, wlp_ref, wr_ref, b_ref, y_ref, yr_ref):
  y_ref[...] = _DOT(h, wlp_ref[...])
  yr_ref[...] = _DOT(h, wr_ref[...]) + b_ref[0:1, :]


def _tc_first(x, diff, rec_prev, hidden_prev, wlp, wr, b):
  def body(x_ref, d_ref, r_ref, hp_ref, wlp_ref, wr_ref, b_ref, y_ref, yr_ref):
    h = jnp.concatenate(
        [x_ref[...], d_ref[...], r_ref[...], hp_ref[...]], axis=1)
    _emit_layer(h, wlp_ref, wr_ref, b_ref, y_ref, yr_ref)

  return pl.pallas_call(
      body,
      grid=(NRB,),
      in_specs=[pl.BlockSpec((RB, 48), lambda i: (i, 0)),
                pl.BlockSpec((RB, 24), lambda i: (i, 0)),
                pl.BlockSpec((RB, 12), lambda i: (i, 0)),
                pl.BlockSpec((RB, 12), lambda i: (i, 0)),
                _WP_SPEC, _W_SPEC, _B_SPEC],
      out_specs=[_Y_SPEC, _H_SPEC],
      out_shape=[jax.ShapeDtypeStruct((N, FP), jnp.float32),
                 jax.ShapeDtypeStruct((N, F), jnp.float32)],
  )(x, diff, rec_prev, hidden_prev, wlp, wr, b)


def _combine(p_ref, cnt_ref, yr_ref):
  seg = p_ref[0, :, 0:F] + p_ref[1, :, 0:F]
  cnt = cnt_ref[0, :, 0:1] + cnt_ref[1, :, 0:1]
  inv = 1.0 / jnp.maximum(cnt, 1.0)
  return jax.nn.relu(seg * inv + yr_ref[...])


def _tc_mid(ps, cntp, yr_prev, wlp, wr, b):
  def body(p_ref, cnt_ref, yrp_ref, wlp_ref, wr_ref, b_ref, y_ref, yr_ref):
    h = _combine(p_ref, cnt_ref, yrp_ref)
    _emit_layer(h, wlp_ref, wr_ref, b_ref, y_ref, yr_ref)

  return pl.pallas_call(
      body,
      grid=(NRB,),
      in_specs=[_P_SPEC, _P_SPEC, _H_SPEC, _WP_SPEC, _W_SPEC, _B_SPEC],
      out_specs=[_Y_SPEC, _H_SPEC],
      out_shape=[jax.ShapeDtypeStruct((N, FP), jnp.float32),
                 jax.ShapeDtypeStruct((N, F), jnp.float32)],
  )(ps, cntp, yr_prev, wlp, wr, b)


def _tc_last_pre(ps, cntp, yr_prev, w3p, b3):
  def body(p_ref, cnt_ref, yrp_ref, w_ref, b_ref, y3_ref, yr3_ref):
    h = _combine(p_ref, cnt_ref, yrp_ref)
    y3 = _DOT(h, w_ref[...])
    y3_ref[...] = y3
    yr3_ref[...] = y3[:, 1:2] + b_ref[0, 0]

  return pl.pallas_call(
      body,
      grid=(NRB,),
      in_specs=[_P_SPEC, _P_SPEC, _H_SPEC, _WP_SPEC,
                pl.BlockSpec(memory_space=pltpu.SMEM)],
      out_specs=[_Y_SPEC, pl.BlockSpec((RB, 1), lambda i: (i, 0))],
      out_shape=[jax.ShapeDtypeStruct((N, FP), jnp.float32),
                 jax.ShapeDtypeStruct((N, 1), jnp.float32)],
  )(ps, cntp, yr_prev, w3p, b3)


def _tc_final(p3, cntp, yr3):
  def body(p3_ref, cnt_ref, yr3_ref, out_ref, util_ref, acc_ref):
    i = pl.program_id(0)

    @pl.when(i == 0)
    def _():
      acc_ref[0] = 0.0

    seg = p3_ref[0, :, 0:1] + p3_ref[1, :, 0:1]
    cnt = cnt_ref[0, :, 0:1] + cnt_ref[1, :, 0:1]
    inv = 1.0 / jnp.maximum(cnt, 1.0)
    o = jax.nn.sigmoid(seg * inv + yr3_ref[...])
    out_ref[...] = o
    acc_ref[0] = acc_ref[0] + jnp.sum(o)
    util_ref[0] = acc_ref[0] * (1.0 / N)

  return pl.pallas_call(
      body,
      grid=(NRB,),
      in_specs=[_P_SPEC, _P_SPEC, pl.BlockSpec((RB, 1), lambda i: (i, 0))],
      out_specs=[pl.BlockSpec((RB, 1), lambda i: (i, 0)),
                 pl.BlockSpec(memory_space=pltpu.SMEM)],
      out_shape=[jax.ShapeDtypeStruct((N, 1), jnp.float32),
                 jax.ShapeDtypeStruct((1,), jnp.float32)],
      scratch_shapes=[pltpu.SMEM((1,), jnp.float32)],
  )(p3, cntp, yr3)


def kernel(x, diff, rec_prev, hidden_prev, edge_index,
           Wl0, Wr0, b0, Wl1, Wr1, b1, Wl2, Wr2, b2, Wl3, Wr3, b3):
  eidx = edge_index.reshape(2, NW, NBLK, EB)

  zc = jnp.zeros((ZBLK, FC), jnp.float32)
  z8 = jnp.zeros((ZBLK, 8), jnp.float32)
  ones8 = jnp.ones((EB, 8), jnp.float32)

  def pad_wl(wl):
    return jnp.concatenate([wl, jnp.zeros((FP - F, F), jnp.float32)], axis=0)

  def pad_b(b):
    return jnp.broadcast_to(b[None, :], (8, F))

  cntp = _sc_count(ones8, eidx, z8)

  y, yr = _tc_first(x, diff, rec_prev, hidden_prev,
                    pad_wl(Wl0), Wr0, pad_b(b0))
  y, cntp = lax.optimization_barrier((y, cntp))
  ps = _sc_segsum_wide(_sc_deint(y, NCHUNK), eidx, zc)

  y, yr = _tc_mid(ps, cntp, yr, pad_wl(Wl1), Wr1, pad_b(b1))
  ps = _sc_segsum_wide(_sc_deint(y, NCHUNK), eidx, zc)

  y, yr = _tc_mid(ps, cntp, yr, pad_wl(Wl2), Wr2, pad_b(b2))
  ps = _sc_segsum_wide(_sc_deint(y, NCHUNK), eidx, zc)

  w3p = jnp.concatenate([Wl3, Wr3, jnp.zeros((FP - 2, F), jnp.float32)],
                        axis=0)
  y3p, yr3 = _tc_last_pre(ps, cntp, yr, w3p, b3.reshape(1, 1))
  p3 = _sc_segsum_narrow(_sc_deint(y3p, 1), eidx, zc)

  out, util = _tc_final(p3, cntp, yr3)
  return out, util[0]


# speedup vs baseline: 9.2110x; 1.0038x over previous
import functools

import jax
import jax.numpy as jnp
from jax import lax
from jax.experimental import pallas as pl
from jax.experimental.pallas import tpu as pltpu
from jax.experimental.pallas import tpu_sc as plsc

N = 50000
E = 800000
F = 96
FP = 128

NCORE = 2
NSUB = 16
NW = NCORE * NSUB
EPW = E // NW
EB = 125
NBLK = EPW // EB
NBUF = 10
ZBLK = 1000
NZB = N // ZBLK
ZPT = -(-NZB // NSUB)

FC = 16
NCHUNK = F // FC

RB = 5000
NRB = N // RB


def _sc_mesh():
  return plsc.VectorSubcoreMesh(core_axis_name="c", subcore_axis_name="s")


_SC_PARAMS = pltpu.CompilerParams(use_tc_tiling_on_sc=False)


def _edge_pass(table_h, src_v, dst_v, rows, acc, gsem, ssem, gather):
  def group(g, carry):
    if gather:
      for b in range(NBUF):
        j = g * NBUF + b

        @pl.when(g > 0)
        def _():
          pltpu.make_async_copy(rows.at[b], acc.at[dst_v.at[j]],
                                ssem.at[b]).wait()

        pltpu.async_copy(table_h.at[src_v.at[j]], rows.at[b], gsem.at[b])
    for b in range(NBUF):
      j = g * NBUF + b
      if gather:
        pltpu.make_async_copy(table_h.at[src_v.at[j]], rows.at[b],
                              gsem.at[b]).wait()
      else:
        @pl.when(g > 0)
        def _():
          pltpu.make_async_copy(rows.at[b], acc.at[dst_v.at[j]],
                                ssem.at[b]).wait()
      pltpu.async_copy(rows.at[b], acc.at[dst_v.at[j]], ssem.at[b], add=True)
    return carry

  lax.fori_loop(0, NBLK // NBUF, group, 0)
  for b in range(NBUF):
    j = NBLK - NBUF + b
    pltpu.make_async_copy(rows.at[b], acc.at[dst_v.at[j]], ssem.at[b]).wait()


def _zero_acc(z_h, acc, sid):
  for t in range(ZPT):
    blk = sid + NSUB * t

    @pl.when(blk < NZB)
    def _():
      pltpu.sync_copy(z_h, acc.at[pl.ds(blk * ZBLK, ZBLK)])


def _writeout(acc, out_h, cid, sid, col):
  w = acc.shape[-1]
  for t in range(ZPT):
    blk = sid + NSUB * t

    @pl.when(blk < NZB)
    def _():
      pltpu.sync_copy(acc.at[pl.ds(blk * ZBLK, ZBLK)],
                      out_h.at[cid, pl.ds(blk * ZBLK, ZBLK), pl.ds(col, w)])


DB = 500
NDB = N // DB
DPT = -(-NDB // NW)


def _sc_deint(y, nchunk):
  width = FP if nchunk > 1 else FC

  @functools.partial(
      pl.kernel,
      out_type=jax.ShapeDtypeStruct((nchunk, N, FC), jnp.float32),
      mesh=_sc_mesh(),
      compiler_params=_SC_PARAMS,
      scratch_types=[
          pltpu.VMEM((2, DB, width), jnp.float32),
          pltpu.SemaphoreType.DMA((2,)),
          pltpu.SemaphoreType.DMA((2,)),
      ],
  )
  def k(y_h, o_h, buf, isem, osem):
    cid = lax.axis_index("c")
    sid = lax.axis_index("s")
    wid = cid * NSUB + sid

    def src_of(t):
      blk = wid + NW * t
      if nchunk > 1:
        return y_h.at[pl.ds(blk * DB, DB)]
      return y_h.at[pl.ds(blk * DB, DB), pl.ds(0, FC)]

    def blk_of(t):
      return wid + NW * t

    def drain_out(t):
      p = t % 2
      for c in range(nchunk):
        pltpu.make_async_copy(buf.at[p].at[:, pl.ds(c * FC, FC)],
                              o_h.at[c, pl.ds(blk_of(t) * DB, DB)],
                              osem.at[p]).wait()

    @pl.when(blk_of(0) < NDB)
    def _():
      pltpu.async_copy(src_of(0), buf.at[0], isem.at[0])
    for t in range(DPT):
      blk = blk_of(t)
      p = t % 2

      @pl.when(blk < NDB)
      def _():
        pltpu.make_async_copy(src_of(t), buf.at[p], isem.at[p]).wait()
        if t >= 1:
          drain_out(t - 1)
        if t + 1 < DPT:
          @pl.when(blk_of(t + 1) < NDB)
          def _():
            pltpu.async_copy(src_of(t + 1), buf.at[1 - p], isem.at[1 - p])
        for c in range(nchunk):
          pltpu.async_copy(buf.at[p].at[:, pl.ds(c * FC, FC)],
                           o_h.at[c, pl.ds(blk * DB, DB)], osem.at[p])
    for t in range(DPT):
      last = (blk_of(t) < NDB)
      if t + 1 < DPT:
        last = last & (blk_of(t + 1) >= NDB)

      @pl.when(last)
      def _():
        drain_out(t)

  return k(y)


def _sc_segsum_wide(y, eidx, zblk):
  @functools.partial(
      pl.kernel,
      out_type=jax.ShapeDtypeStruct((NCORE, N, FP), jnp.float32),
      mesh=_sc_mesh(),
      compiler_params=_SC_PARAMS,
      scratch_types=[
          pltpu.VMEM((NBLK, EB), jnp.int32),
          pltpu.VMEM((NBLK, EB), jnp.int32),
          pltpu.VMEM((NBUF, EB, FC), jnp.float32),
          pltpu.VMEM_SHARED((N, FC), jnp.float32),
          pltpu.SemaphoreType.DMA((NBUF,)),
          pltpu.SemaphoreType.DMA((NBUF,)),
      ],
  )
  def k(y_h, e_h, z_h, o_h, src_v, dst_v, rows, acc, gsem, ssem):
    cid = lax.axis_index("c")
    sid = lax.axis_index("s")
    wid = cid * NSUB + sid
    pltpu.sync_copy(e_h.at[0, wid], src_v)
    pltpu.sync_copy(e_h.at[1, wid], dst_v)
    for c in range(NCHUNK):
      table = y_h.at[c]
      _zero_acc(z_h, acc, sid)
      plsc.subcore_barrier()
      _edge_pass(table, src_v, dst_v, rows, acc, gsem, ssem, gather=True)
      plsc.subcore_barrier()
      _writeout(acc, o_h, cid, sid, c * FC)
      plsc.subcore_barrier()

  return k(y, eidx, zblk)


def _sc_segsum_narrow(y, eidx, zblk):
  @functools.partial(
      pl.kernel,
      out_type=jax.ShapeDtypeStruct((NCORE, N, FP), jnp.float32),
      mesh=_sc_mesh(),
      compiler_params=_SC_PARAMS,
      scratch_types=[
          pltpu.VMEM((NBLK, EB), jnp.int32),
          pltpu.VMEM((NBLK, EB), jnp.int32),
          pltpu.VMEM((NBUF, EB, FC), jnp.float32),
          pltpu.VMEM_SHARED((N, FC), jnp.float32),
          pltpu.SemaphoreType.DMA((NBUF,)),
          pltpu.SemaphoreType.DMA((NBUF,)),
      ],
  )
  def k(y_h, e_h, z_h, o_h, src_v, dst_v, rows, acc, gsem, ssem):
    cid = lax.axis_index("c")
    sid = lax.axis_index("s")
    wid = cid * NSUB + sid
    pltpu.sync_copy(e_h.at[0, wid], src_v)
    pltpu.sync_copy(e_h.at[1, wid], dst_v)
    table = y_h.at[0]
    _zero_acc(z_h, acc, sid)
    plsc.subcore_barrier()
    _edge_pass(table, src_v, dst_v, rows, acc, gsem, ssem, gather=True)
    plsc.subcore_barrier()
    _writeout(acc, o_h, cid, sid, 0)

  return k(y, eidx, zblk)


def _sc_count(ones_rows, eidx, zblk):
  @functools.partial(
      pl.kernel,
      out_type=jax.ShapeDtypeStruct((NCORE, N, FP), jnp.float32),
      mesh=_sc_mesh(),
      compiler_params=_SC_PARAMS,
      scratch_types=[
          pltpu.VMEM((NBLK, EB), jnp.int32),
          pltpu.VMEM((NBUF, EB, 8), jnp.float32),
          pltpu.VMEM_SHARED((N, 8), jnp.float32),
          pltpu.SemaphoreType.DMA((NBUF,)),
          pltpu.SemaphoreType.DMA((NBUF,)),
      ],
  )
  def k(ones_h, e_h, z_h, o_h, dst_v, rows, acc, gsem, ssem):
    cid = lax.axis_index("c")
    sid = lax.axis_index("s")
    wid = cid * NSUB + sid
    pltpu.sync_copy(e_h.at[1, wid], dst_v)
    for b in range(NBUF):
      pltpu.sync_copy(ones_h, rows.at[b])
    _zero_acc(z_h, acc, sid)
    plsc.subcore_barrier()
    _edge_pass(None, None, dst_v, rows, acc, gsem, ssem, gather=False)
    plsc.subcore_barrier()
    _writeout(acc, o_h, cid, sid, 0)

  return k(ones_rows, eidx, zblk)


_DOT = functools.partial(
    lax.dot_general,
    dimension_numbers=(((1,), (1,)), ((), ())),
    preferred_element_type=jnp.float32,
)

_WP_SPEC = pl.BlockSpec((FP, F), lambda i: (0, 0))
_W_SPEC = pl.BlockSpec((F, F), lambda i: (0, 0))
_B_SPEC = pl.BlockSpec((8, F), lambda i: (0, 0))
_H_SPEC = pl.BlockSpec((RB, F), lambda i: (i, 0))
_Y_SPEC = pl.BlockSpec((RB, FP), lambda i: (i, 0))
_P_SPEC = pl.BlockSpec((NCORE, RB, FP), lambda i: (0, i, 0))


def _emit_layer(h, wlp_ref, wr_ref, b_ref, y_ref, yr_ref):
  y_ref[...] = _DOT(h, wlp_ref[...])
  yr_ref[...] = _DOT(h, wr_ref[...]) + b_ref[0:1, :]


def _tc_first(x, diff, rec_prev, hidden_prev, wlp, wr, b):
  def body(x_ref, d_ref, r_ref, hp_ref, wlp_ref, wr_ref, b_ref, y_ref, yr_ref):
    h = jnp.concatenate(
        [x_ref[...], d_ref[...], r_ref[...], hp_ref[...]], axis=1)
    _emit_layer(h, wlp_ref, wr_ref, b_ref, y_ref, yr_ref)

  return pl.pallas_call(
      body,
      grid=(NRB,),
      in_specs=[pl.BlockSpec((RB, 48), lambda i: (i, 0)),
                pl.BlockSpec((RB, 24), lambda i: (i, 0)),
                pl.BlockSpec((RB, 12), lambda i: (i, 0)),
                pl.BlockSpec((RB, 12), lambda i: (i, 0)),
                _WP_SPEC, _W_SPEC, _B_SPEC],
      out_specs=[_Y_SPEC, _H_SPEC],
      out_shape=[jax.ShapeDtypeStruct((N, FP), jnp.float32),
                 jax.ShapeDtypeStruct((N, F), jnp.float32)],
  )(x, diff, rec_prev, hidden_prev, wlp, wr, b)


def _combine(p_ref, cnt_ref, yr_ref):
  seg = p_ref[0, :, 0:F] + p_ref[1, :, 0:F]
  cnt = cnt_ref[0, :, 0:1] + cnt_ref[1, :, 0:1]
  inv = 1.0 / jnp.maximum(cnt, 1.0)
  return jax.nn.relu(seg * inv + yr_ref[...])


def _tc_mid(ps, cntp, yr_prev, wlp, wr, b):
  def body(p_ref, cnt_ref, yrp_ref, wlp_ref, wr_ref, b_ref, y_ref, yr_ref):
    h = _combine(p_ref, cnt_ref, yrp_ref)
    _emit_layer(h, wlp_ref, wr_ref, b_ref, y_ref, yr_ref)

  return pl.pallas_call(
      body,
      grid=(NRB,),
      in_specs=[_P_SPEC, _P_SPEC, _H_SPEC, _WP_SPEC, _W_SPEC, _B_SPEC],
      out_specs=[_Y_SPEC, _H_SPEC],
      out_shape=[jax.ShapeDtypeStruct((N, FP), jnp.float32),
                 jax.ShapeDtypeStruct((N, F), jnp.float32)],
  )(ps, cntp, yr_prev, wlp, wr, b)


def _tc_last_pre(ps, cntp, yr_prev, w3p, b3):
  def body(p_ref, cnt_ref, yrp_ref, w_ref, b_ref, y3_ref, yr3_ref):
    h = _combine(p_ref, cnt_ref, yrp_ref)
    y3 = _DOT(h, w_ref[...])
    y3_ref[...] = y3
    yr3_ref[...] = y3[:, 1:2] + b_ref[0, 0]

  return pl.pallas_call(
      body,
      grid=(NRB,),
      in_specs=[_P_SPEC, _P_SPEC, _H_SPEC, _WP_SPEC,
                pl.BlockSpec(memory_space=pltpu.SMEM)],
      out_specs=[_Y_SPEC, pl.BlockSpec((RB, 1), lambda i: (i, 0))],
      out_shape=[jax.ShapeDtypeStruct((N, FP), jnp.float32),
                 jax.ShapeDtypeStruct((N, 1), jnp.float32)],
  )(ps, cntp, yr_prev, w3p, b3)


def _tc_final(p3, cntp, yr3):
  def body(p3_ref, cnt_ref, yr3_ref, out_ref, util_ref, acc_ref):
    i = pl.program_id(0)

    @pl.when(i == 0)
    def _():
      acc_ref[0] = 0.0

    seg = p3_ref[0, :, 0:1] + p3_ref[1, :, 0:1]
    cnt = cnt_ref[0, :, 0:1] + cnt_ref[1, :, 0:1]
    inv = 1.0 / jnp.maximum(cnt, 1.0)
    o = jax.nn.sigmoid(seg * inv + yr3_ref[...])
    out_ref[...] = o
    acc_ref[0] = acc_ref[0] + jnp.sum(o)
    util_ref[0] = acc_ref[0] * (1.0 / N)

  return pl.pallas_call(
      body,
      grid=(NRB,),
      in_specs=[_P_SPEC, _P_SPEC, pl.BlockSpec((RB, 1), lambda i: (i, 0))],
      out_specs=[pl.BlockSpec((RB, 1), lambda i: (i, 0)),
                 pl.BlockSpec(memory_space=pltpu.SMEM)],
      out_shape=[jax.ShapeDtypeStruct((N, 1), jnp.float32),
                 jax.ShapeDtypeStruct((1,), jnp.float32)],
      scratch_shapes=[pltpu.SMEM((1,), jnp.float32)],
  )(p3, cntp, yr3)


def kernel(x, diff, rec_prev, hidden_prev, edge_index,
           Wl0, Wr0, b0, Wl1, Wr1, b1, Wl2, Wr2, b2, Wl3, Wr3, b3):
  eidx = edge_index.reshape(2, NW, NBLK, EB)

  zc = jnp.zeros((ZBLK, FC), jnp.float32)
  z8 = jnp.zeros((ZBLK, 8), jnp.float32)
  ones8 = jnp.ones((EB, 8), jnp.float32)

  def pad_wl(wl):
    return jnp.concatenate([wl, jnp.zeros((FP - F, F), jnp.float32)], axis=0)

  def pad_b(b):
    return jnp.broadcast_to(b[None, :], (8, F))

  cntp = _sc_count(ones8, eidx, z8)

  y, yr = _tc_first(x, diff, rec_prev, hidden_prev,
                    pad_wl(Wl0), Wr0, pad_b(b0))
  y, cntp = lax.optimization_barrier((y, cntp))
  ps = _sc_segsum_wide(_sc_deint(y, NCHUNK), eidx, zc)

  y, yr = _tc_mid(ps, cntp, yr, pad_wl(Wl1), Wr1, pad_b(b1))
  ps = _sc_segsum_wide(_sc_deint(y, NCHUNK), eidx, zc)

  y, yr = _tc_mid(ps, cntp, yr, pad_wl(Wl2), Wr2, pad_b(b2))
  ps = _sc_segsum_wide(_sc_deint(y, NCHUNK), eidx, zc)

  w3p = jnp.concatenate([Wl3, Wr3, jnp.zeros((FP - 2, F), jnp.float32)],
                        axis=0)
  y3p, yr3 = _tc_last_pre(ps, cntp, yr, w3p, b3.reshape(1, 1))
  p3 = _sc_segsum_narrow(_sc_deint(y3p, 1), eidx, zc)

  out, util = _tc_final(p3, cntp, yr3)
  return out, util[0]
